# baseline (speedup 1.0000x reference)
import functools

import jax
import jax.numpy as jnp
from jax.experimental import pallas as pl

N = 10000
NUM_HEAD = 1


def _matmul_att_kernel(z_ref, w_ref, atti_ref, attj_ref, h_ref, ai_ref, aj_ref):
    h = jnp.dot(z_ref[...], w_ref[...], preferred_element_type=jnp.float32)
    h_ref[...] = h
    ai_ref[...] = jnp.sum(h * atti_ref[...], axis=1, keepdims=True)
    aj_ref[...] = jnp.sum(h * attj_ref[...], axis=1, keepdims=True)


def _project(z, W, att):
    n, d = z.shape
    emb = W.shape[0]
    att = att.reshape(2 * emb)
    atti = att[:emb].reshape(1, emb)
    attj = att[emb:].reshape(1, emb)
    blk = 1000
    grid = (n // blk,)
    h, ai, aj = pl.pallas_call(
        _matmul_att_kernel,
        grid=grid,
        in_specs=[
            pl.BlockSpec((blk, d), lambda i: (i, 0)),
            pl.BlockSpec((d, emb), lambda i: (0, 0)),
            pl.BlockSpec((1, emb), lambda i: (0, 0)),
            pl.BlockSpec((1, emb), lambda i: (0, 0)),
        ],
        out_specs=[
            pl.BlockSpec((blk, emb), lambda i: (i, 0)),
            pl.BlockSpec((blk, 1), lambda i: (i, 0)),
            pl.BlockSpec((blk, 1), lambda i: (i, 0)),
        ],
        out_shape=[
            jax.ShapeDtypeStruct((n, emb), jnp.float32),
            jax.ShapeDtypeStruct((n, 1), jnp.float32),
            jax.ShapeDtypeStruct((n, 1), jnp.float32),
        ],
    )(z, W.T, atti, attj)
    return h, ai[:, 0], aj[:, 0]


def _edge_phase(h, ai, aj, src, dst, negative_slope=0.2):
    n = h.shape[0]
    alpha = ai[dst] + aj[src]
    alpha = jnp.where(alpha >= 0, alpha, negative_slope * alpha)
    amax = jax.ops.segment_max(alpha, src, num_segments=n)
    p = jnp.exp(alpha - amax[src])
    s = jax.ops.segment_sum(p, src, num_segments=n)
    w = p / (s[src] + 1e-16)
    msg = h[src] * w[:, None]
    return jax.ops.segment_sum(msg, dst, num_segments=n)


def _bn_leaky(h, g, be, eps=1e-5, slope=0.1):
    m = h.mean(0)
    v = h.var(0)
    z = (h - m) / jnp.sqrt(v + eps) * g + be
    return jnp.where(z >= 0, z, slope * z)


def _gat_layer(z, W, att, bias, g, be, src, dst):
    h, ai, aj = _project(z, W, att)
    agg = _edge_phase(h, ai, aj, src, dst)
    out = jax.nn.relu(agg + bias)
    return _bn_leaky(out, g, be)


@jax.jit
def _forward_impl(x, edge_index, drug_index, W1, att1, b1, W2, att2, b2,
                  W3, att3, b3, g1, be1, g2, be2, g3, be3, P1, P2):
    n = x.shape[0]
    loops = jnp.arange(n, dtype=edge_index.dtype)
    src = jnp.concatenate([edge_index[0], loops])
    dst = jnp.concatenate([edge_index[1], loops])

    z = _gat_layer(x, W1, att1, b1, g1, be1, src, dst)
    z = _gat_layer(z, W2, att2, b2, g2, be2, src, dst)
    z = _gat_layer(z, W3, att3, b3, g3, be3, src, dst)

    di = drug_index.reshape(-1, 2)
    ia = (di[:, 0] - 1) % n
    ib = (di[:, 1] - 1) % n
    a = z[ia]
    bb = z[ib]
    M = ((a @ P1) @ P2) @ P1.T
    return jnp.sum(M * bb, axis=1, keepdims=True)


def kernel(x, edge_index, drug_index, W1, att1, b1, W2, att2, b2, W3, att3,
           b3, g1, be1, g2, be2, g3, be3, P1, P2):
    return _forward_impl(x, edge_index, drug_index, W1, att1, b1, W2, att2,
                         b2, W3, att3, b3, g1, be1, g2, be2, g3, be3, P1, P2)


# speedup vs baseline: 6.7433x; 6.7433x over previous
import dataclasses
import functools

import jax
import jax.numpy as jnp
from jax import lax
from jax.experimental import pallas as pl
from jax.experimental.pallas import tpu as pltpu
from jax.experimental.pallas import tpu_sc as plsc

N = 10000
NPAD = 10240
EHAT = 330000
NTILE = 16
CH3 = 128
T = 20736
EPAD = NTILE * T
NCHUNK3 = T // CH3
SLICE = NPAD // NTILE


BLK = 1024
NBLK = NPAD // BLK


def _proj_kernel(z_ref, w_ref, atti_ref, attj_ref,
                 hl_ref, hr_ref, ai_ref, aj_ref):
    h = jnp.dot(z_ref[...], w_ref[...], preferred_element_type=jnp.float32)
    hl_ref[...] = h[:, :128]
    hr_ref[...] = h[:, 128:]
    ai_ref[...] = jnp.sum(h * atti_ref[...], axis=1, keepdims=True)
    aj_ref[...] = jnp.sum(h * attj_ref[...], axis=1, keepdims=True)


def _proj_bn_kernel(o_ref, st_ref, g_ref, be_ref, w_ref, atti_ref, attj_ref,
                    hl_ref, hr_ref, ai_ref, aj_ref):
    m = st_ref[0, :] / N
    v = st_ref[1, :] / N - m * m
    z = (o_ref[...] - m) / jnp.sqrt(v + 1e-5) * g_ref[...] + be_ref[...]
    z = jnp.where(z >= 0, z, 0.1 * z)
    h = jnp.dot(z, w_ref[...], preferred_element_type=jnp.float32)
    hl_ref[...] = h[:, :128]
    hr_ref[...] = h[:, 128:]
    ai_ref[...] = jnp.sum(h * atti_ref[...], axis=1, keepdims=True)
    aj_ref[...] = jnp.sum(h * attj_ref[...], axis=1, keepdims=True)


def _project(z, W, att, stats=None, g=None, be=None):
    n, d = z.shape
    emb = W.shape[0]
    attf = att.reshape(2 * emb)
    atti = attf[:emb].reshape(1, emb)
    attj = attf[emb:].reshape(1, emb)
    out_shape = [
        jax.ShapeDtypeStruct((n, 128), jnp.float32),
        jax.ShapeDtypeStruct((n, 128), jnp.float32),
        jax.ShapeDtypeStruct((n, 1), jnp.float32),
        jax.ShapeDtypeStruct((n, 1), jnp.float32),
    ]
    out_specs = [
        pl.BlockSpec((BLK, 128), lambda i: (i, 0)),
        pl.BlockSpec((BLK, 128), lambda i: (i, 0)),
        pl.BlockSpec((BLK, 1), lambda i: (i, 0)),
        pl.BlockSpec((BLK, 1), lambda i: (i, 0)),
    ]
    if stats is None:
        hl, hr, ai, aj = pl.pallas_call(
            _proj_kernel,
            grid=(n // BLK,),
            in_specs=[
                pl.BlockSpec((BLK, d), lambda i: (i, 0)),
                pl.BlockSpec((d, emb), lambda i: (0, 0)),
                pl.BlockSpec((1, emb), lambda i: (0, 0)),
                pl.BlockSpec((1, emb), lambda i: (0, 0)),
            ],
            out_specs=out_specs,
            out_shape=out_shape,
        )(z, W.T, atti, attj)
    else:
        hl, hr, ai, aj = pl.pallas_call(
            _proj_bn_kernel,
            grid=(n // BLK,),
            in_specs=[
                pl.BlockSpec((BLK, d), lambda i: (i, 0)),
                pl.BlockSpec((2, emb), lambda i: (0, 0)),
                pl.BlockSpec((1, emb), lambda i: (0, 0)),
                pl.BlockSpec((1, emb), lambda i: (0, 0)),
                pl.BlockSpec((d, emb), lambda i: (0, 0)),
                pl.BlockSpec((1, emb), lambda i: (0, 0)),
                pl.BlockSpec((1, emb), lambda i: (0, 0)),
            ],
            out_specs=out_specs,
            out_shape=out_shape,
        )(z, stats, g.reshape(1, emb), be.reshape(1, emb), W.T, atti, attj)
    return hl, hr, ai.reshape(n), aj.reshape(n)


def _stats_kernel(al_ref, ar_ref, b_ref, out_ref, st_ref, acc_ref):
    i = pl.program_id(0)
    o = jnp.concatenate([al_ref[...], ar_ref[...]], axis=1) + b_ref[...]
    o = jnp.maximum(o, 0.0)
    out_ref[...] = o
    rowid = lax.broadcasted_iota(jnp.int32, o.shape, 0) + i * BLK
    om = jnp.where(rowid < N, o, 0.0)
    part = jnp.stack([jnp.sum(om, axis=0), jnp.sum(om * om, axis=0)])

    @pl.when(i == 0)
    def _():
        acc_ref[...] = jnp.zeros_like(acc_ref)

    acc_ref[...] += part

    @pl.when(i == NBLK - 1)
    def _():
        st_ref[...] = acc_ref[...]


def _stats_relu(aggl, aggr, bias):
    emb = bias.shape[0]
    out, st = pl.pallas_call(
        _stats_kernel,
        grid=(NBLK,),
        in_specs=[
            pl.BlockSpec((BLK, 128), lambda i: (i, 0)),
            pl.BlockSpec((BLK, 128), lambda i: (i, 0)),
            pl.BlockSpec((1, emb), lambda i: (0, 0)),
        ],
        out_specs=[
            pl.BlockSpec((BLK, emb), lambda i: (i, 0)),
            pl.BlockSpec((2, emb), lambda i: (0, 0)),
        ],
        out_shape=[
            jax.ShapeDtypeStruct((NPAD, emb), jnp.float32),
            jax.ShapeDtypeStruct((2, emb), jnp.float32),
        ],
        scratch_shapes=[pltpu.VMEM((2, emb), jnp.float32)],
    )(aggl, aggr, bias.reshape(1, emb))
    return out, st


def _bn_apply_kernel(o_ref, st_ref, g_ref, be_ref, z_ref):
    m = st_ref[0, :] / N
    v = st_ref[1, :] / N - m * m
    z = (o_ref[...] - m) / jnp.sqrt(v + 1e-5) * g_ref[...] + be_ref[...]
    z_ref[...] = jnp.where(z >= 0, z, 0.1 * z)


def _bn_apply(out, stats, g, be):
    emb = g.shape[0]
    return pl.pallas_call(
        _bn_apply_kernel,
        grid=(NBLK,),
        in_specs=[
            pl.BlockSpec((BLK, emb), lambda i: (i, 0)),
            pl.BlockSpec((2, emb), lambda i: (0, 0)),
            pl.BlockSpec((1, emb), lambda i: (0, 0)),
            pl.BlockSpec((1, emb), lambda i: (0, 0)),
        ],
        out_specs=pl.BlockSpec((BLK, emb), lambda i: (i, 0)),
        out_shape=jax.ShapeDtypeStruct((NPAD, emb), jnp.float32),
    )(out, stats, g.reshape(1, emb), be.reshape(1, emb))


def _dec_kernel(a_ref, bb_ref, p1_ref, p2_ref, p1t_ref, y_ref):
    t = jnp.dot(a_ref[...], p1_ref[...], preferred_element_type=jnp.float32)
    t = jnp.dot(t, p2_ref[...], preferred_element_type=jnp.float32)
    mm = jnp.dot(t, p1t_ref[...], preferred_element_type=jnp.float32)
    y_ref[...] = jnp.sum(mm * bb_ref[...], axis=1, keepdims=True)


def _decode(a, bb, P1, P2):
    b = a.shape[0]
    return pl.pallas_call(
        _dec_kernel,
        out_shape=jax.ShapeDtypeStruct((b, 1), jnp.float32),
    )(a, bb, P1, P2, P1.T)


def _seg_combine(sv, val, ks_buf, vs_buf, is_max):
    ks, vs = plsc.sort_key_val(sv, val)
    ks_buf[...] = ks
    it = lax.iota(jnp.int32, 16)
    for sh in (1, 2, 4, 8):
        vs_buf[...] = vs
        idx = jnp.minimum(it + sh, 15)
        kg = plsc.load_gather(ks_buf, [idx])
        vg = plsc.load_gather(vs_buf, [idx])
        same = (kg == ks) & (it + sh <= 15)
        if is_max:
            vs = jnp.where(same, jnp.maximum(vs, vg), vs)
        else:
            vs = jnp.where(same, vs + vg, vs)
    prev = plsc.load_gather(ks_buf, [jnp.maximum(it - 1, 0)])
    first = (it == 0) | (ks != prev)
    return ks, vs, first


def _sc_compiler_params():
    cp = pltpu.CompilerParams()
    if "needs_layout_passes" in pltpu.CompilerParams.__dataclass_fields__:
        cp = dataclasses.replace(cp, needs_layout_passes=False)
    return cp


def _sc_weights(ai, aj, src16, dst16):
    mesh = plsc.VectorSubcoreMesh(core_axis_name="c", subcore_axis_name="s",
                                  num_cores=1, num_subcores=NTILE)

    @functools.partial(
        pl.kernel,
        out_type=jax.ShapeDtypeStruct((NTILE, T), jnp.float32),
        mesh=mesh,
        compiler_params=_sc_compiler_params(),
        scratch_types=[
            pltpu.VMEM((NPAD,), jnp.float32),
            pltpu.VMEM((NPAD,), jnp.float32),
            pltpu.VMEM((NPAD,), jnp.float32),
            pltpu.VMEM((NPAD,), jnp.float32),
            pltpu.VMEM((T,), jnp.int32),
            pltpu.VMEM((T,), jnp.int32),
            pltpu.VMEM((T,), jnp.float32),
            pltpu.VMEM((16,), jnp.int32),
            pltpu.VMEM((16,), jnp.float32),
            pltpu.VMEM((SLICE,), jnp.float32),
            pltpu.VMEM((SLICE,), jnp.float32),
            pltpu.VMEM_SHARED((NTILE, NPAD), jnp.float32),
            pltpu.VMEM_SHARED((NPAD,), jnp.float32),
        ],
    )
    def k(ai_hbm, aj_hbm, src_hbm, dst_hbm, w_hbm,
          ai_v, aj_v, m_v, s_v, src_v, dst_v, w_v,
          ks_buf, vs_buf, tmp, red, part, full):
        s = lax.axis_index("s")
        base = s * SLICE

        pltpu.sync_copy(ai_hbm, ai_v)
        pltpu.sync_copy(aj_hbm, aj_v)
        pltpu.sync_copy(src_hbm.at[s], src_v)
        pltpu.sync_copy(dst_hbm.at[s], dst_v)

        zero16 = jnp.zeros((16,), jnp.float32)
        neginf16 = jnp.full((16,), -jnp.inf, jnp.float32)

        @pl.loop(0, NPAD, step=16)
        def _(i):
            m_v[pl.ds(i, 16)] = neginf16
            s_v[pl.ds(i, 16)] = zero16

        def alpha_at(e0):
            sv = src_v[pl.ds(e0, 16)]
            dv = dst_v[pl.ds(e0, 16)]
            a1 = plsc.load_gather(ai_v, [dv])
            a2 = plsc.load_gather(aj_v, [sv])
            al = a1 + a2
            al = jnp.where(al >= 0.0, al, 0.2 * al)
            return sv, al

        @pl.loop(0, T, step=16)
        def _(e0):
            sv, al = alpha_at(e0)
            ks, vs, first = _seg_combine(sv, al, ks_buf, vs_buf, is_max=True)
            cur = plsc.load_gather(m_v, [ks])
            plsc.store_scatter(m_v, [ks], jnp.maximum(cur, vs), mask=first)

        pltpu.sync_copy(m_v, part.at[s])
        plsc.subcore_barrier()
        pltpu.sync_copy(part.at[0, pl.ds(base, SLICE)], red)
        for t in range(1, NTILE):
            pltpu.sync_copy(part.at[t, pl.ds(base, SLICE)], tmp)

            @pl.loop(0, SLICE, step=16)
            def _(i):
                red[pl.ds(i, 16)] = jnp.maximum(red[pl.ds(i, 16)],
                                                tmp[pl.ds(i, 16)])

        pltpu.sync_copy(red, full.at[pl.ds(base, SLICE)])
        plsc.subcore_barrier()
        pltpu.sync_copy(full, m_v)

        @pl.loop(0, T, step=16)
        def _(e0):
            sv, al = alpha_at(e0)
            am = plsc.load_gather(m_v, [sv])
            p = jnp.exp(al - am)
            ks, vs, first = _seg_combine(sv, p, ks_buf, vs_buf, is_max=False)
            cur = plsc.load_gather(s_v, [ks])
            plsc.store_scatter(s_v, [ks], cur + vs, mask=first)

        pltpu.sync_copy(s_v, part.at[s])
        plsc.subcore_barrier()
        pltpu.sync_copy(part.at[0, pl.ds(base, SLICE)], red)
        for t in range(1, NTILE):
            pltpu.sync_copy(part.at[t, pl.ds(base, SLICE)], tmp)

            @pl.loop(0, SLICE, step=16)
            def _(i):
                red[pl.ds(i, 16)] = red[pl.ds(i, 16)] + tmp[pl.ds(i, 16)]

        @pl.loop(0, SLICE, step=16)
        def _(i):
            red[pl.ds(i, 16)] = 1.0 / (red[pl.ds(i, 16)] + 1e-16)

        pltpu.sync_copy(red, full.at[pl.ds(base, SLICE)])
        plsc.subcore_barrier()
        pltpu.sync_copy(full, s_v)

        @pl.loop(0, T, step=16)
        def _(e0):
            sv, al = alpha_at(e0)
            am = plsc.load_gather(m_v, [sv])
            rr = plsc.load_gather(s_v, [sv])
            w_v[pl.ds(e0, 16)] = jnp.exp(al - am) * rr

        pltpu.sync_copy(w_v, w_hbm.at[s])

    return k(ai, aj, src16, dst16)


def _sc_aggregate(hhalf, w16, src16, dst16):
    mesh = plsc.VectorSubcoreMesh(core_axis_name="c", subcore_axis_name="s",
                                  num_cores=1, num_subcores=NTILE)

    @functools.partial(
        pl.kernel,
        out_type=jax.ShapeDtypeStruct((NPAD, 128), jnp.float32),
        mesh=mesh,
        compiler_params=_sc_compiler_params(),
        scratch_types=[
            pltpu.VMEM((CH3, 128), jnp.float32),
            pltpu.VMEM((CH3,), jnp.float32),
            pltpu.VMEM((CH3,), jnp.int32),
            pltpu.VMEM((CH3,), jnp.int32),
            pltpu.VMEM_SHARED((NPAD, 128), jnp.float32),
        ],
    )
    def k(h_hbm, w_hbm, src_hbm, dst_hbm, out_hbm,
          rows, w_buf, dstidx, srcg, acc):
        s = lax.axis_index("s")
        base = s * SLICE

        zero16 = jnp.zeros((16,), jnp.float32)

        @pl.loop(0, CH3)
        def _(rr):
            for q in range(8):
                rows[rr, pl.ds(q * 16, 16)] = zero16

        for b in range(SLICE // CH3):
            pltpu.sync_copy(rows, acc.at[pl.ds(base + b * CH3, CH3)])

        plsc.subcore_barrier()

        @pl.loop(0, NCHUNK3)
        def _(j):
            e0 = j * CH3
            pltpu.sync_copy(w_hbm.at[s, pl.ds(e0, CH3)], w_buf)
            pltpu.sync_copy(dst_hbm.at[s, pl.ds(e0, CH3)], dstidx)
            pltpu.sync_copy(src_hbm.at[s, pl.ds(e0, CH3)], srcg)
            pltpu.sync_copy(h_hbm.at[srcg], rows)

            @pl.loop(0, CH3, step=16)
            def _(r0):
                wv = w_buf[pl.ds(r0, 16)]
                for q1 in range(16):
                    wb = jnp.full((16,), wv[q1], jnp.float32)
                    for q2 in range(8):
                        sl = (r0 + q1, pl.ds(q2 * 16, 16))
                        rows[sl] = rows[sl] * wb

            pltpu.sync_copy(rows, acc.at[dstidx], add=True)

        plsc.subcore_barrier()
        pltpu.sync_copy(acc.at[pl.ds(base, SLICE)],
                        out_hbm.at[pl.ds(base, SLICE)])

    return k(hhalf, w16, src16, dst16)


def _xla_weights(ai, aj, src16, dst16):
    src = src16.reshape(-1)
    dst = dst16.reshape(-1)
    al = ai[dst] + aj[src]
    al = jnp.where(al >= 0, al, 0.2 * al)
    amax = jax.ops.segment_max(al, src, num_segments=NPAD)
    p = jnp.exp(al - amax[src])
    s = jax.ops.segment_sum(p, src, num_segments=NPAD)
    w = p / (s[src] + 1e-16)
    return w.reshape(NTILE, T)


def _gat_layer(z, W, att, bias, src16, dst16, stats=None, g=None, be=None):
    hl, hr, ai, aj = _project(z, W, att, stats, g, be)
    aip = jnp.pad(ai, (0, NPAD - ai.shape[0])) if ai.shape[0] != NPAD else ai
    ajp = jnp.pad(aj, (0, NPAD - aj.shape[0])) if aj.shape[0] != NPAD else aj
    w16 = _sc_weights(aip, ajp, src16, dst16)
    aggl = _sc_aggregate(hl, w16, src16, dst16)
    aggr = _sc_aggregate(hr, w16, src16, dst16)
    return aggl, aggr


@jax.jit
def _forward_impl(x, edge_index, drug_index, W1, att1, b1, W2, att2, b2,
                  W3, att3, b3, g1, be1, g2, be2, g3, be3, P1, P2):
    loops = jnp.arange(N, dtype=edge_index.dtype)
    pad = jnp.full((EPAD - EHAT,), NPAD - 1, dtype=edge_index.dtype)
    src = jnp.concatenate([edge_index[0], loops, pad]).astype(jnp.int32)
    dst = jnp.concatenate([edge_index[1], loops, pad]).astype(jnp.int32)
    src16 = src.reshape(NTILE, T)
    dst16 = dst.reshape(NTILE, T)

    xp = jnp.pad(x, ((0, NPAD - N), (0, 0)))

    al, ar = _gat_layer(xp, W1, att1, b1, src16, dst16)
    out, st = _stats_relu(al, ar, b1)
    al, ar = _gat_layer(out, W2, att2, b2, src16, dst16, st, g1, be1)
    out, st = _stats_relu(al, ar, b2)
    al, ar = _gat_layer(out, W3, att3, b3, src16, dst16, st, g2, be2)
    out, st = _stats_relu(al, ar, b3)
    z3 = _bn_apply(out, st, g3, be3)

    di = drug_index.reshape(-1, 2)
    ia = (di[:, 0] - 1) % N
    ib = (di[:, 1] - 1) % N
    a = z3[ia]
    bb = z3[ib]
    return _decode(a, bb, P1, P2)


def kernel(x, edge_index, drug_index, W1, att1, b1, W2, att2, b2, W3, att3,
           b3, g1, be1, g2, be2, g3, be3, P1, P2):
    return _forward_impl(x, edge_index, drug_index, W1, att1, b1, W2, att2,
                         b2, W3, att3, b3, g1, be1, g2, be2, g3, be3, P1, P2)


# speedup vs baseline: 13.3054x; 1.9731x over previous
import dataclasses
import functools

import jax
import jax.numpy as jnp
from jax import lax
from jax.experimental import pallas as pl
from jax.experimental.pallas import tpu as pltpu
from jax.experimental.pallas import tpu_sc as plsc

N = 10000
NPAD = 10240
EHAT = 330000
NTILE = 16
CH3 = 256
ZB = 128
T = 20736
EPAD = NTILE * T
NCHUNK3 = T // CH3
SLICE = NPAD // NTILE


BLK = 1024
NBLK = NPAD // BLK


def _proj_kernel(z_ref, w_ref, atti_ref, attj_ref,
                 hl_ref, hr_ref, ai_ref, aj_ref):
    h = jnp.dot(z_ref[...], w_ref[...], preferred_element_type=jnp.float32)
    hl_ref[...] = h[:, :128]
    hr_ref[...] = h[:, 128:]
    ai_ref[...] = jnp.sum(h * atti_ref[...], axis=1, keepdims=True)
    aj_ref[...] = jnp.sum(h * attj_ref[...], axis=1, keepdims=True)


def _proj_bn_kernel(o_ref, st_ref, g_ref, be_ref, w_ref, atti_ref, attj_ref,
                    hl_ref, hr_ref, ai_ref, aj_ref):
    m = st_ref[0, :] / N
    v = st_ref[1, :] / N - m * m
    z = (o_ref[...] - m) / jnp.sqrt(v + 1e-5) * g_ref[...] + be_ref[...]
    z = jnp.where(z >= 0, z, 0.1 * z)
    h = jnp.dot(z, w_ref[...], preferred_element_type=jnp.float32)
    hl_ref[...] = h[:, :128]
    hr_ref[...] = h[:, 128:]
    ai_ref[...] = jnp.sum(h * atti_ref[...], axis=1, keepdims=True)
    aj_ref[...] = jnp.sum(h * attj_ref[...], axis=1, keepdims=True)


def _project(z, W, att, stats=None, g=None, be=None):
    n, d = z.shape
    emb = W.shape[0]
    attf = att.reshape(2 * emb)
    atti = attf[:emb].reshape(1, emb)
    attj = attf[emb:].reshape(1, emb)
    out_shape = [
        jax.ShapeDtypeStruct((n, 128), jnp.float32),
        jax.ShapeDtypeStruct((n, 128), jnp.float32),
        jax.ShapeDtypeStruct((n, 1), jnp.float32),
        jax.ShapeDtypeStruct((n, 1), jnp.float32),
    ]
    out_specs = [
        pl.BlockSpec((BLK, 128), lambda i: (i, 0)),
        pl.BlockSpec((BLK, 128), lambda i: (i, 0)),
        pl.BlockSpec((BLK, 1), lambda i: (i, 0)),
        pl.BlockSpec((BLK, 1), lambda i: (i, 0)),
    ]
    if stats is None:
        hl, hr, ai, aj = pl.pallas_call(
            _proj_kernel,
            grid=(n // BLK,),
            in_specs=[
                pl.BlockSpec((BLK, d), lambda i: (i, 0)),
                pl.BlockSpec((d, emb), lambda i: (0, 0)),
                pl.BlockSpec((1, emb), lambda i: (0, 0)),
                pl.BlockSpec((1, emb), lambda i: (0, 0)),
            ],
            out_specs=out_specs,
            out_shape=out_shape,
        )(z, W.T, atti, attj)
    else:
        hl, hr, ai, aj = pl.pallas_call(
            _proj_bn_kernel,
            grid=(n // BLK,),
            in_specs=[
                pl.BlockSpec((BLK, d), lambda i: (i, 0)),
                pl.BlockSpec((2, emb), lambda i: (0, 0)),
                pl.BlockSpec((1, emb), lambda i: (0, 0)),
                pl.BlockSpec((1, emb), lambda i: (0, 0)),
                pl.BlockSpec((d, emb), lambda i: (0, 0)),
                pl.BlockSpec((1, emb), lambda i: (0, 0)),
                pl.BlockSpec((1, emb), lambda i: (0, 0)),
            ],
            out_specs=out_specs,
            out_shape=out_shape,
        )(z, stats, g.reshape(1, emb), be.reshape(1, emb), W.T, atti, attj)
    return hl, hr, ai.reshape(n), aj.reshape(n)


def _stats_kernel(al_ref, ar_ref, b_ref, out_ref, st_ref, acc_ref):
    i = pl.program_id(0)
    o = jnp.concatenate([al_ref[...], ar_ref[...]], axis=1) + b_ref[...]
    o = jnp.maximum(o, 0.0)
    out_ref[...] = o
    rowid = lax.broadcasted_iota(jnp.int32, o.shape, 0) + i * BLK
    om = jnp.where(rowid < N, o, 0.0)
    part = jnp.stack([jnp.sum(om, axis=0), jnp.sum(om * om, axis=0)])

    @pl.when(i == 0)
    def _():
        acc_ref[...] = jnp.zeros_like(acc_ref)

    acc_ref[...] += part

    @pl.when(i == NBLK - 1)
    def _():
        st_ref[...] = acc_ref[...]


def _stats_relu(aggl, aggr, bias):
    emb = bias.shape[0]
    out, st = pl.pallas_call(
        _stats_kernel,
        grid=(NBLK,),
        in_specs=[
            pl.BlockSpec((BLK, 128), lambda i: (i, 0)),
            pl.BlockSpec((BLK, 128), lambda i: (i, 0)),
            pl.BlockSpec((1, emb), lambda i: (0, 0)),
        ],
        out_specs=[
            pl.BlockSpec((BLK, emb), lambda i: (i, 0)),
            pl.BlockSpec((2, emb), lambda i: (0, 0)),
        ],
        out_shape=[
            jax.ShapeDtypeStruct((NPAD, emb), jnp.float32),
            jax.ShapeDtypeStruct((2, emb), jnp.float32),
        ],
        scratch_shapes=[pltpu.VMEM((2, emb), jnp.float32)],
    )(aggl, aggr, bias.reshape(1, emb))
    return out, st


def _bn_apply_kernel(o_ref, st_ref, g_ref, be_ref, z_ref):
    m = st_ref[0, :] / N
    v = st_ref[1, :] / N - m * m
    z = (o_ref[...] - m) / jnp.sqrt(v + 1e-5) * g_ref[...] + be_ref[...]
    z_ref[...] = jnp.where(z >= 0, z, 0.1 * z)


def _bn_apply(out, stats, g, be):
    emb = g.shape[0]
    return pl.pallas_call(
        _bn_apply_kernel,
        grid=(NBLK,),
        in_specs=[
            pl.BlockSpec((BLK, emb), lambda i: (i, 0)),
            pl.BlockSpec((2, emb), lambda i: (0, 0)),
            pl.BlockSpec((1, emb), lambda i: (0, 0)),
            pl.BlockSpec((1, emb), lambda i: (0, 0)),
        ],
        out_specs=pl.BlockSpec((BLK, emb), lambda i: (i, 0)),
        out_shape=jax.ShapeDtypeStruct((NPAD, emb), jnp.float32),
    )(out, stats, g.reshape(1, emb), be.reshape(1, emb))


def _dec_kernel(a_ref, bb_ref, p1_ref, p2_ref, p1t_ref, y_ref):
    t = jnp.dot(a_ref[...], p1_ref[...], preferred_element_type=jnp.float32)
    t = jnp.dot(t, p2_ref[...], preferred_element_type=jnp.float32)
    mm = jnp.dot(t, p1t_ref[...], preferred_element_type=jnp.float32)
    y_ref[...] = jnp.sum(mm * bb_ref[...], axis=1, keepdims=True)


def _decode(a, bb, P1, P2):
    b = a.shape[0]
    return pl.pallas_call(
        _dec_kernel,
        out_shape=jax.ShapeDtypeStruct((b, 1), jnp.float32),
    )(a, bb, P1, P2, P1.T)


def _seg_combine(sv, val, ks_buf, vs_buf, is_max):
    ks, vs = plsc.sort_key_val(sv, val)
    ks_buf[...] = ks
    it = lax.iota(jnp.int32, 16)
    for sh in (1, 2, 4, 8):
        vs_buf[...] = vs
        idx = jnp.minimum(it + sh, 15)
        kg = plsc.load_gather(ks_buf, [idx])
        vg = plsc.load_gather(vs_buf, [idx])
        same = (kg == ks) & (it + sh <= 15)
        if is_max:
            vs = jnp.where(same, jnp.maximum(vs, vg), vs)
        else:
            vs = jnp.where(same, vs + vg, vs)
    prev = plsc.load_gather(ks_buf, [jnp.maximum(it - 1, 0)])
    first = (it == 0) | (ks != prev)
    return ks, vs, first


def _sc_compiler_params():
    cp = pltpu.CompilerParams()
    if "needs_layout_passes" in pltpu.CompilerParams.__dataclass_fields__:
        cp = dataclasses.replace(cp, needs_layout_passes=False)
    return cp


def _sc_weights(ai, aj, src16, dst16):
    mesh = plsc.VectorSubcoreMesh(core_axis_name="c", subcore_axis_name="s",
                                  num_cores=1, num_subcores=NTILE)

    @functools.partial(
        pl.kernel,
        out_type=jax.ShapeDtypeStruct((NTILE, T), jnp.float32),
        mesh=mesh,
        compiler_params=_sc_compiler_params(),
        scratch_types=[
            pltpu.VMEM((NPAD,), jnp.float32),
            pltpu.VMEM((NPAD,), jnp.float32),
            pltpu.VMEM((NPAD,), jnp.float32),
            pltpu.VMEM((NPAD,), jnp.float32),
            pltpu.VMEM((T,), jnp.int32),
            pltpu.VMEM((T,), jnp.int32),
            pltpu.VMEM((T,), jnp.float32),
            pltpu.VMEM((16,), jnp.int32),
            pltpu.VMEM((16,), jnp.float32),
            pltpu.VMEM((SLICE,), jnp.float32),
            pltpu.VMEM((SLICE,), jnp.float32),
            pltpu.VMEM_SHARED((NTILE, NPAD), jnp.float32),
            pltpu.VMEM_SHARED((NPAD,), jnp.float32),
        ],
    )
    def k(ai_hbm, aj_hbm, src_hbm, dst_hbm, w_hbm,
          ai_v, aj_v, m_v, s_v, src_v, dst_v, w_v,
          ks_buf, vs_buf, tmp, red, part, full):
        s = lax.axis_index("s")
        base = s * SLICE

        pltpu.sync_copy(ai_hbm, ai_v)
        pltpu.sync_copy(aj_hbm, aj_v)
        pltpu.sync_copy(src_hbm.at[s], src_v)
        pltpu.sync_copy(dst_hbm.at[s], dst_v)

        zero16 = jnp.zeros((16,), jnp.float32)
        neginf16 = jnp.full((16,), -jnp.inf, jnp.float32)

        @pl.loop(0, NPAD, step=16)
        def _(i):
            m_v[pl.ds(i, 16)] = neginf16
            s_v[pl.ds(i, 16)] = zero16

        def alpha_at(e0):
            sv = src_v[pl.ds(e0, 16)]
            dv = dst_v[pl.ds(e0, 16)]
            a1 = plsc.load_gather(ai_v, [dv])
            a2 = plsc.load_gather(aj_v, [sv])
            al = a1 + a2
            al = jnp.where(al >= 0.0, al, 0.2 * al)
            return sv, al

        @pl.loop(0, T, step=16)
        def _(e0):
            sv, al = alpha_at(e0)
            ks, vs, first = _seg_combine(sv, al, ks_buf, vs_buf, is_max=True)
            cur = plsc.load_gather(m_v, [ks])
            plsc.store_scatter(m_v, [ks], jnp.maximum(cur, vs), mask=first)

        pltpu.sync_copy(m_v, part.at[s])
        plsc.subcore_barrier()
        pltpu.sync_copy(part.at[0, pl.ds(base, SLICE)], red)
        for t in range(1, NTILE):
            pltpu.sync_copy(part.at[t, pl.ds(base, SLICE)], tmp)

            @pl.loop(0, SLICE, step=16)
            def _(i):
                red[pl.ds(i, 16)] = jnp.maximum(red[pl.ds(i, 16)],
                                                tmp[pl.ds(i, 16)])

        pltpu.sync_copy(red, full.at[pl.ds(base, SLICE)])
        plsc.subcore_barrier()
        pltpu.sync_copy(full, m_v)

        @pl.loop(0, T, step=16)
        def _(e0):
            sv, al = alpha_at(e0)
            am = plsc.load_gather(m_v, [sv])
            p = jnp.exp(al - am)
            ks, vs, first = _seg_combine(sv, p, ks_buf, vs_buf, is_max=False)
            cur = plsc.load_gather(s_v, [ks])
            plsc.store_scatter(s_v, [ks], cur + vs, mask=first)

        pltpu.sync_copy(s_v, part.at[s])
        plsc.subcore_barrier()
        pltpu.sync_copy(part.at[0, pl.ds(base, SLICE)], red)
        for t in range(1, NTILE):
            pltpu.sync_copy(part.at[t, pl.ds(base, SLICE)], tmp)

            @pl.loop(0, SLICE, step=16)
            def _(i):
                red[pl.ds(i, 16)] = red[pl.ds(i, 16)] + tmp[pl.ds(i, 16)]

        @pl.loop(0, SLICE, step=16)
        def _(i):
            red[pl.ds(i, 16)] = 1.0 / (red[pl.ds(i, 16)] + 1e-16)

        pltpu.sync_copy(red, full.at[pl.ds(base, SLICE)])
        plsc.subcore_barrier()
        pltpu.sync_copy(full, s_v)

        @pl.loop(0, T, step=16)
        def _(e0):
            sv, al = alpha_at(e0)
            am = plsc.load_gather(m_v, [sv])
            rr = plsc.load_gather(s_v, [sv])
            w_v[pl.ds(e0, 16)] = jnp.exp(al - am) * rr

        pltpu.sync_copy(w_v, w_hbm.at[s])

    return k(ai, aj, src16, dst16)


def _sc_aggregate2(hstack, w16, src16, dst16):
    mesh = plsc.VectorSubcoreMesh(core_axis_name="c", subcore_axis_name="s",
                                  num_cores=2, num_subcores=NTILE)

    @functools.partial(
        pl.kernel,
        out_type=jax.ShapeDtypeStruct((2 * NPAD, 128), jnp.float32),
        mesh=mesh,
        compiler_params=_sc_compiler_params(),
        scratch_types=[
            pltpu.VMEM((CH3, 128), jnp.float32),
            pltpu.VMEM((CH3,), jnp.float32),
            pltpu.VMEM((CH3,), jnp.int32),
            pltpu.VMEM((CH3,), jnp.int32),
            pltpu.VMEM_SHARED((NPAD, 128), jnp.float32),
        ],
    )
    def k(h_hbm, w_hbm, src_hbm, dst_hbm, out_hbm,
          rows, w_buf, dstidx, srcg, acc):
        c = lax.axis_index("c")
        s = lax.axis_index("s")
        base = s * SLICE
        hoff = c * NPAD

        zero16 = jnp.zeros((16,), jnp.float32)

        @pl.loop(0, ZB)
        def _(rr):
            for q in range(8):
                rows[rr, pl.ds(q * 16, 16)] = zero16

        for b in range(SLICE // ZB):
            pltpu.sync_copy(rows.at[pl.ds(0, ZB)],
                            acc.at[pl.ds(base + b * ZB, ZB)])

        plsc.subcore_barrier()

        @pl.loop(0, NCHUNK3)
        def _(j):
            e0 = j * CH3
            pltpu.sync_copy(w_hbm.at[s, pl.ds(e0, CH3)], w_buf)
            pltpu.sync_copy(dst_hbm.at[s, pl.ds(e0, CH3)], dstidx)
            pltpu.sync_copy(src_hbm.at[s, pl.ds(e0, CH3)], srcg)

            off16 = jnp.full((16,), hoff, jnp.int32)

            @pl.loop(0, CH3, step=16)
            def _(i):
                srcg[pl.ds(i, 16)] = srcg[pl.ds(i, 16)] + off16

            pltpu.sync_copy(h_hbm.at[srcg], rows)

            @pl.loop(0, CH3, step=16)
            def _(r0):
                wv = w_buf[pl.ds(r0, 16)]
                for q1 in range(16):
                    wb = jnp.full((16,), wv[q1], jnp.float32)
                    for q2 in range(8):
                        sl = (r0 + q1, pl.ds(q2 * 16, 16))
                        rows[sl] = rows[sl] * wb

            pltpu.sync_copy(rows, acc.at[dstidx], add=True)

        plsc.subcore_barrier()
        pltpu.sync_copy(acc.at[pl.ds(base, SLICE)],
                        out_hbm.at[pl.ds(hoff + base, SLICE)])

    return k(hstack, w16, src16, dst16)


def _gat_layer(z, W, att, bias, src16, dst16, stats=None, g=None, be=None):
    hl, hr, ai, aj = _project(z, W, att, stats, g, be)
    w16 = _sc_weights(ai, aj, src16, dst16)
    agg = _sc_aggregate2(jnp.concatenate([hl, hr], axis=0),
                         w16, src16, dst16)
    return agg[:NPAD], agg[NPAD:]


@jax.jit
def _forward_impl(x, edge_index, drug_index, W1, att1, b1, W2, att2, b2,
                  W3, att3, b3, g1, be1, g2, be2, g3, be3, P1, P2):
    loops = jnp.arange(N, dtype=edge_index.dtype)
    pad = jnp.full((EPAD - EHAT,), NPAD - 1, dtype=edge_index.dtype)
    src = jnp.concatenate([edge_index[0], loops, pad]).astype(jnp.int32)
    dst = jnp.concatenate([edge_index[1], loops, pad]).astype(jnp.int32)
    src16 = src.reshape(NTILE, T)
    dst16 = dst.reshape(NTILE, T)

    xp = jnp.pad(x, ((0, NPAD - N), (0, 0)))

    al, ar = _gat_layer(xp, W1, att1, b1, src16, dst16)
    out, st = _stats_relu(al, ar, b1)
    al, ar = _gat_layer(out, W2, att2, b2, src16, dst16, st, g1, be1)
    out, st = _stats_relu(al, ar, b2)
    al, ar = _gat_layer(out, W3, att3, b3, src16, dst16, st, g2, be2)
    out, st = _stats_relu(al, ar, b3)
    z3 = _bn_apply(out, st, g3, be3)

    di = drug_index.reshape(-1, 2)
    ia = (di[:, 0] - 1) % N
    ib = (di[:, 1] - 1) % N
    a = z3[ia]
    bb = z3[ib]
    return _decode(a, bb, P1, P2)


def kernel(x, edge_index, drug_index, W1, att1, b1, W2, att2, b2, W3, att3,
           b3, g1, be1, g2, be2, g3, be3, P1, P2):
    return _forward_impl(x, edge_index, drug_index, W1, att1, b1, W2, att2,
                         b2, W3, att3, b3, g1, be1, g2, be2, g3, be3, P1, P2)


# speedup vs baseline: 17.5262x; 1.3172x over previous
import dataclasses
import functools

import jax
import jax.numpy as jnp
from jax import lax
from jax.experimental import pallas as pl
from jax.experimental.pallas import tpu as pltpu
from jax.experimental.pallas import tpu_sc as plsc

N = 10000
NPAD = 10240
EHAT = 330000
NTILE = 16
CH3 = 128
ZB = 128
T = 20736
SB = 2304
NSB = T // SB
NCB = SB // CH3
EPAD = NTILE * T
NCHUNK3 = T // CH3
SLICE = NPAD // NTILE


BLK = 1024
NBLK = NPAD // BLK


def _proj_kernel(z_ref, w_ref, atti_ref, attj_ref,
                 hl_ref, hr_ref, ai_ref, aj_ref):
    h = jnp.dot(z_ref[...], w_ref[...], preferred_element_type=jnp.float32)
    hl_ref[...] = h[:, :128]
    hr_ref[...] = h[:, 128:]
    ai_ref[...] = jnp.sum(h * atti_ref[...], axis=1, keepdims=True)
    aj_ref[...] = jnp.sum(h * attj_ref[...], axis=1, keepdims=True)


def _proj_bn_kernel(o_ref, st_ref, g_ref, be_ref, w_ref, atti_ref, attj_ref,
                    hl_ref, hr_ref, ai_ref, aj_ref):
    m = st_ref[0, :] / N
    v = st_ref[1, :] / N - m * m
    z = (o_ref[...] - m) / jnp.sqrt(v + 1e-5) * g_ref[...] + be_ref[...]
    z = jnp.where(z >= 0, z, 0.1 * z)
    h = jnp.dot(z, w_ref[...], preferred_element_type=jnp.float32)
    hl_ref[...] = h[:, :128]
    hr_ref[...] = h[:, 128:]
    ai_ref[...] = jnp.sum(h * atti_ref[...], axis=1, keepdims=True)
    aj_ref[...] = jnp.sum(h * attj_ref[...], axis=1, keepdims=True)


def _project(z, W, att, stats=None, g=None, be=None):
    n, d = z.shape
    emb = W.shape[0]
    attf = att.reshape(2 * emb)
    atti = attf[:emb].reshape(1, emb)
    attj = attf[emb:].reshape(1, emb)
    out_shape = [
        jax.ShapeDtypeStruct((n, 128), jnp.float32),
        jax.ShapeDtypeStruct((n, 128), jnp.float32),
        jax.ShapeDtypeStruct((n, 1), jnp.float32),
        jax.ShapeDtypeStruct((n, 1), jnp.float32),
    ]
    out_specs = [
        pl.BlockSpec((BLK, 128), lambda i: (i, 0)),
        pl.BlockSpec((BLK, 128), lambda i: (i, 0)),
        pl.BlockSpec((BLK, 1), lambda i: (i, 0)),
        pl.BlockSpec((BLK, 1), lambda i: (i, 0)),
    ]
    if stats is None:
        hl, hr, ai, aj = pl.pallas_call(
            _proj_kernel,
            grid=(n // BLK,),
            in_specs=[
                pl.BlockSpec((BLK, d), lambda i: (i, 0)),
                pl.BlockSpec((d, emb), lambda i: (0, 0)),
                pl.BlockSpec((1, emb), lambda i: (0, 0)),
                pl.BlockSpec((1, emb), lambda i: (0, 0)),
            ],
            out_specs=out_specs,
            out_shape=out_shape,
        )(z, W.T, atti, attj)
    else:
        hl, hr, ai, aj = pl.pallas_call(
            _proj_bn_kernel,
            grid=(n // BLK,),
            in_specs=[
                pl.BlockSpec((BLK, d), lambda i: (i, 0)),
                pl.BlockSpec((2, emb), lambda i: (0, 0)),
                pl.BlockSpec((1, emb), lambda i: (0, 0)),
                pl.BlockSpec((1, emb), lambda i: (0, 0)),
                pl.BlockSpec((d, emb), lambda i: (0, 0)),
                pl.BlockSpec((1, emb), lambda i: (0, 0)),
                pl.BlockSpec((1, emb), lambda i: (0, 0)),
            ],
            out_specs=out_specs,
            out_shape=out_shape,
        )(z, stats, g.reshape(1, emb), be.reshape(1, emb), W.T, atti, attj)
    return hl, hr, ai.reshape(n), aj.reshape(n)


def _stats_kernel(al_ref, ar_ref, b_ref, out_ref, st_ref, acc_ref):
    i = pl.program_id(0)
    o = jnp.concatenate([al_ref[...], ar_ref[...]], axis=1) + b_ref[...]
    o = jnp.maximum(o, 0.0)
    out_ref[...] = o
    rowid = lax.broadcasted_iota(jnp.int32, o.shape, 0) + i * BLK
    om = jnp.where(rowid < N, o, 0.0)
    part = jnp.stack([jnp.sum(om, axis=0), jnp.sum(om * om, axis=0)])

    @pl.when(i == 0)
    def _():
        acc_ref[...] = jnp.zeros_like(acc_ref)

    acc_ref[...] += part

    @pl.when(i == NBLK - 1)
    def _():
        st_ref[...] = acc_ref[...]


def _stats_relu(aggl, aggr, bias):
    emb = bias.shape[0]
    out, st = pl.pallas_call(
        _stats_kernel,
        grid=(NBLK,),
        in_specs=[
            pl.BlockSpec((BLK, 128), lambda i: (i, 0)),
            pl.BlockSpec((BLK, 128), lambda i: (i, 0)),
            pl.BlockSpec((1, emb), lambda i: (0, 0)),
        ],
        out_specs=[
            pl.BlockSpec((BLK, emb), lambda i: (i, 0)),
            pl.BlockSpec((2, emb), lambda i: (0, 0)),
        ],
        out_shape=[
            jax.ShapeDtypeStruct((NPAD, emb), jnp.float32),
            jax.ShapeDtypeStruct((2, emb), jnp.float32),
        ],
        scratch_shapes=[pltpu.VMEM((2, emb), jnp.float32)],
    )(aggl, aggr, bias.reshape(1, emb))
    return out, st


def _bn_apply_kernel(o_ref, st_ref, g_ref, be_ref, z_ref):
    m = st_ref[0, :] / N
    v = st_ref[1, :] / N - m * m
    z = (o_ref[...] - m) / jnp.sqrt(v + 1e-5) * g_ref[...] + be_ref[...]
    z_ref[...] = jnp.where(z >= 0, z, 0.1 * z)


def _bn_apply(out, stats, g, be):
    emb = g.shape[0]
    return pl.pallas_call(
        _bn_apply_kernel,
        grid=(NBLK,),
        in_specs=[
            pl.BlockSpec((BLK, emb), lambda i: (i, 0)),
            pl.BlockSpec((2, emb), lambda i: (0, 0)),
            pl.BlockSpec((1, emb), lambda i: (0, 0)),
            pl.BlockSpec((1, emb), lambda i: (0, 0)),
        ],
        out_specs=pl.BlockSpec((BLK, emb), lambda i: (i, 0)),
        out_shape=jax.ShapeDtypeStruct((NPAD, emb), jnp.float32),
    )(out, stats, g.reshape(1, emb), be.reshape(1, emb))


def _dec_kernel(a_ref, bb_ref, p1_ref, p2_ref, p1t_ref, y_ref):
    t = jnp.dot(a_ref[...], p1_ref[...], preferred_element_type=jnp.float32)
    t = jnp.dot(t, p2_ref[...], preferred_element_type=jnp.float32)
    mm = jnp.dot(t, p1t_ref[...], preferred_element_type=jnp.float32)
    y_ref[...] = jnp.sum(mm * bb_ref[...], axis=1, keepdims=True)


def _decode(a, bb, P1, P2):
    b = a.shape[0]
    return pl.pallas_call(
        _dec_kernel,
        out_shape=jax.ShapeDtypeStruct((b, 1), jnp.float32),
    )(a, bb, P1, P2, P1.T)


def _seg_combine(sv, val, ks_buf, vs_buf, is_max):
    ks, vs = plsc.sort_key_val(sv, val)
    ks_buf[...] = ks
    it = lax.iota(jnp.int32, 16)
    for sh in (1, 2, 4, 8):
        vs_buf[...] = vs
        idx = jnp.minimum(it + sh, 15)
        kg = plsc.load_gather(ks_buf, [idx])
        vg = plsc.load_gather(vs_buf, [idx])
        same = (kg == ks) & (it + sh <= 15)
        if is_max:
            vs = jnp.where(same, jnp.maximum(vs, vg), vs)
        else:
            vs = jnp.where(same, vs + vg, vs)
    prev = plsc.load_gather(ks_buf, [jnp.maximum(it - 1, 0)])
    first = (it == 0) | (ks != prev)
    return ks, vs, first


def _sc_compiler_params():
    cp = pltpu.CompilerParams()
    if "needs_layout_passes" in pltpu.CompilerParams.__dataclass_fields__:
        cp = dataclasses.replace(cp, needs_layout_passes=False)
    return cp


def _sc_weights(ai, aj, src16, dst16):
    mesh = plsc.VectorSubcoreMesh(core_axis_name="c", subcore_axis_name="s",
                                  num_cores=1, num_subcores=NTILE)

    @functools.partial(
        pl.kernel,
        out_type=jax.ShapeDtypeStruct((NTILE, T), jnp.float32),
        mesh=mesh,
        compiler_params=_sc_compiler_params(),
        scratch_types=[
            pltpu.VMEM((NPAD,), jnp.float32),
            pltpu.VMEM((NPAD,), jnp.float32),
            pltpu.VMEM((NPAD,), jnp.float32),
            pltpu.VMEM((NPAD,), jnp.float32),
            pltpu.VMEM((T,), jnp.int32),
            pltpu.VMEM((T,), jnp.int32),
            pltpu.VMEM((T,), jnp.float32),
            pltpu.VMEM((16,), jnp.int32),
            pltpu.VMEM((16,), jnp.float32),
            pltpu.VMEM((SLICE,), jnp.float32),
            pltpu.VMEM((SLICE,), jnp.float32),
            pltpu.VMEM_SHARED((NTILE, NPAD), jnp.float32),
            pltpu.VMEM_SHARED((NPAD,), jnp.float32),
        ],
    )
    def k(ai_hbm, aj_hbm, src_hbm, dst_hbm, w_hbm,
          ai_v, aj_v, m_v, s_v, src_v, dst_v, w_v,
          ks_buf, vs_buf, tmp, red, part, full):
        s = lax.axis_index("s")
        base = s * SLICE

        pltpu.sync_copy(ai_hbm, ai_v)
        pltpu.sync_copy(aj_hbm, aj_v)
        pltpu.sync_copy(src_hbm.at[s], src_v)
        pltpu.sync_copy(dst_hbm.at[s], dst_v)

        zero16 = jnp.zeros((16,), jnp.float32)
        neginf16 = jnp.full((16,), -jnp.inf, jnp.float32)

        @pl.loop(0, NPAD, step=16)
        def _(i):
            m_v[pl.ds(i, 16)] = neginf16
            s_v[pl.ds(i, 16)] = zero16

        def alpha_at(e0):
            sv = src_v[pl.ds(e0, 16)]
            dv = dst_v[pl.ds(e0, 16)]
            a1 = plsc.load_gather(ai_v, [dv])
            a2 = plsc.load_gather(aj_v, [sv])
            al = a1 + a2
            al = jnp.where(al >= 0.0, al, 0.2 * al)
            return sv, al

        @pl.loop(0, T, step=16)
        def _(e0):
            sv, al = alpha_at(e0)
            ks, vs, first = _seg_combine(sv, al, ks_buf, vs_buf, is_max=True)
            cur = plsc.load_gather(m_v, [ks])
            plsc.store_scatter(m_v, [ks], jnp.maximum(cur, vs), mask=first)

        pltpu.sync_copy(m_v, part.at[s])
        plsc.subcore_barrier()
        pltpu.sync_copy(part.at[0, pl.ds(base, SLICE)], red)
        for t in range(1, NTILE):
            pltpu.sync_copy(part.at[t, pl.ds(base, SLICE)], tmp)

            @pl.loop(0, SLICE, step=16)
            def _(i):
                red[pl.ds(i, 16)] = jnp.maximum(red[pl.ds(i, 16)],
                                                tmp[pl.ds(i, 16)])

        pltpu.sync_copy(red, full.at[pl.ds(base, SLICE)])
        plsc.subcore_barrier()
        pltpu.sync_copy(full, m_v)

        @pl.loop(0, T, step=16)
        def _(e0):
            sv, al = alpha_at(e0)
            am = plsc.load_gather(m_v, [sv])
            p = jnp.exp(al - am)
            ks, vs, first = _seg_combine(sv, p, ks_buf, vs_buf, is_max=False)
            cur = plsc.load_gather(s_v, [ks])
            plsc.store_scatter(s_v, [ks], cur + vs, mask=first)

        pltpu.sync_copy(s_v, part.at[s])
        plsc.subcore_barrier()
        pltpu.sync_copy(part.at[0, pl.ds(base, SLICE)], red)
        for t in range(1, NTILE):
            pltpu.sync_copy(part.at[t, pl.ds(base, SLICE)], tmp)

            @pl.loop(0, SLICE, step=16)
            def _(i):
                red[pl.ds(i, 16)] = red[pl.ds(i, 16)] + tmp[pl.ds(i, 16)]

        @pl.loop(0, SLICE, step=16)
        def _(i):
            red[pl.ds(i, 16)] = 1.0 / (red[pl.ds(i, 16)] + 1e-16)

        pltpu.sync_copy(red, full.at[pl.ds(base, SLICE)])
        plsc.subcore_barrier()
        pltpu.sync_copy(full, s_v)

        @pl.loop(0, T, step=16)
        def _(e0):
            sv, al = alpha_at(e0)
            am = plsc.load_gather(m_v, [sv])
            rr = plsc.load_gather(s_v, [sv])
            w_v[pl.ds(e0, 16)] = jnp.exp(al - am) * rr

        pltpu.sync_copy(w_v, w_hbm.at[s])

    return k(ai, aj, src16, dst16)


def _sc_aggregate2(hstack, w16, src16, dst16):
    mesh = plsc.VectorSubcoreMesh(core_axis_name="c", subcore_axis_name="s",
                                  num_cores=2, num_subcores=NTILE)

    @functools.partial(
        pl.kernel,
        out_type=jax.ShapeDtypeStruct((2 * NPAD, 128), jnp.float32),
        mesh=mesh,
        compiler_params=_sc_compiler_params(),
        scratch_types=[
            pltpu.VMEM((CH3, 128), jnp.float32),
            pltpu.VMEM((CH3, 128), jnp.float32),
            pltpu.VMEM((SB,), jnp.float32),
            pltpu.VMEM((SB,), jnp.int32),
            pltpu.VMEM((SB,), jnp.int32),
            pltpu.SemaphoreType.DMA,
            pltpu.SemaphoreType.DMA,
            pltpu.VMEM_SHARED((NPAD, 128), jnp.float32),
        ],
    )
    def k(h_hbm, w_hbm, src_hbm, dst_hbm, out_hbm,
          rows_a, rows_b, w_t, dst_t, src_t, sem_a, sem_b, acc):
        c = lax.axis_index("c")
        s = lax.axis_index("s")
        base = s * SLICE
        hoff = c * NPAD

        zero16 = jnp.zeros((16,), jnp.float32)

        @pl.loop(0, ZB)
        def _(rr):
            for q in range(8):
                rows_a[rr, pl.ds(q * 16, 16)] = zero16

        for b in range(SLICE // ZB):
            pltpu.sync_copy(rows_a.at[pl.ds(0, ZB)],
                            acc.at[pl.ds(base + b * ZB, ZB)])

        plsc.subcore_barrier()

        off16 = jnp.full((16,), hoff, jnp.int32)

        def gather(e0, rows, sem):
            pltpu.make_async_copy(h_hbm.at[src_t.at[pl.ds(e0, CH3)]],
                                  rows, sem).start()

        def finish(e0, rows, sem):
            pltpu.make_async_copy(h_hbm.at[src_t.at[pl.ds(e0, CH3)]],
                                  rows, sem).wait()

            @pl.loop(0, CH3, step=16)
            def _(r0):
                wv = w_t[pl.ds(e0 + r0, 16)]
                for q1 in range(16):
                    wb = jnp.full((16,), wv[q1], jnp.float32)
                    for q2 in range(8):
                        sl = (r0 + q1, pl.ds(q2 * 16, 16))
                        rows[sl] = rows[sl] * wb

            pltpu.sync_copy(rows, acc.at[dst_t.at[pl.ds(e0, CH3)]], add=True)

        @pl.loop(0, NSB)
        def _(bi):
            b0 = bi * SB
            pltpu.sync_copy(w_hbm.at[s, pl.ds(b0, SB)], w_t)
            pltpu.sync_copy(dst_hbm.at[s, pl.ds(b0, SB)], dst_t)
            pltpu.sync_copy(src_hbm.at[s, pl.ds(b0, SB)], src_t)

            @pl.loop(0, SB, step=16)
            def _(i):
                src_t[pl.ds(i, 16)] = src_t[pl.ds(i, 16)] + off16

            gather(0, rows_a, sem_a)

            @pl.loop(0, NCB, step=2)
            def _(j):
                e0 = j * CH3
                gather(e0 + CH3, rows_b, sem_b)
                finish(e0, rows_a, sem_a)

                @pl.when(j + 2 < NCB)
                def _():
                    gather(e0 + 2 * CH3, rows_a, sem_a)

                finish(e0 + CH3, rows_b, sem_b)

        plsc.subcore_barrier()
        pltpu.sync_copy(acc.at[pl.ds(base, SLICE)],
                        out_hbm.at[pl.ds(hoff + base, SLICE)])

    return k(hstack, w16, src16, dst16)


def _gat_layer(z, W, att, bias, src16, dst16, stats=None, g=None, be=None):
    hl, hr, ai, aj = _project(z, W, att, stats, g, be)
    w16 = _sc_weights(ai, aj, src16, dst16)
    agg = _sc_aggregate2(jnp.concatenate([hl, hr], axis=0),
                         w16, src16, dst16)
    return agg[:NPAD], agg[NPAD:]


@jax.jit
def _forward_impl(x, edge_index, drug_index, W1, att1, b1, W2, att2, b2,
                  W3, att3, b3, g1, be1, g2, be2, g3, be3, P1, P2):
    loops = jnp.arange(N, dtype=edge_index.dtype)
    pad = jnp.full((EPAD - EHAT,), NPAD - 1, dtype=edge_index.dtype)
    src = jnp.concatenate([edge_index[0], loops, pad]).astype(jnp.int32)
    dst = jnp.concatenate([edge_index[1], loops, pad]).astype(jnp.int32)
    src16 = src.reshape(NTILE, T)
    dst16 = dst.reshape(NTILE, T)

    xp = jnp.pad(x, ((0, NPAD - N), (0, 0)))

    al, ar = _gat_layer(xp, W1, att1, b1, src16, dst16)
    out, st = _stats_relu(al, ar, b1)
    al, ar = _gat_layer(out, W2, att2, b2, src16, dst16, st, g1, be1)
    out, st = _stats_relu(al, ar, b2)
    al, ar = _gat_layer(out, W3, att3, b3, src16, dst16, st, g2, be2)
    out, st = _stats_relu(al, ar, b3)
    z3 = _bn_apply(out, st, g3, be3)

    di = drug_index.reshape(-1, 2)
    ia = (di[:, 0] - 1) % N
    ib = (di[:, 1] - 1) % N
    a = z3[ia]
    bb = z3[ib]
    return _decode(a, bb, P1, P2)


def kernel(x, edge_index, drug_index, W1, att1, b1, W2, att2, b2, W3, att3,
           b3, g1, be1, g2, be2, g3, be3, P1, P2):
    return _forward_impl(x, edge_index, drug_index, W1, att1, b1, W2, att2,
                         b2, W3, att3, b3, g1, be1, g2, be2, g3, be3, P1, P2)


# speedup vs baseline: 17.7447x; 1.0125x over previous
import dataclasses
import functools

import jax
import jax.numpy as jnp
from jax import lax
from jax.experimental import pallas as pl
from jax.experimental.pallas import tpu as pltpu
from jax.experimental.pallas import tpu_sc as plsc

N = 10000
NPAD = 10240
EHAT = 330000
NTILE = 16
CH3 = 128
ZB = 128
T = 20736
SB = 2304
NSB = T // SB
NCB = SB // CH3
EPAD = NTILE * T
NCHUNK3 = T // CH3
SLICE = NPAD // NTILE


BLK = 1024
NBLK = NPAD // BLK


def _proj_kernel(z_ref, w_ref, atti_ref, attj_ref,
                 hl_ref, hr_ref, ai_ref, aj_ref):
    h = jnp.dot(z_ref[...], w_ref[...], preferred_element_type=jnp.float32)
    hl_ref[...] = h[:, :128]
    hr_ref[...] = h[:, 128:]
    ai_ref[...] = jnp.sum(h * atti_ref[...], axis=1, keepdims=True)
    aj_ref[...] = jnp.sum(h * attj_ref[...], axis=1, keepdims=True)


def _proj_bn_kernel(o_ref, st_ref, g_ref, be_ref, w_ref, atti_ref, attj_ref,
                    hl_ref, hr_ref, ai_ref, aj_ref):
    m = st_ref[0, :] / N
    v = st_ref[1, :] / N - m * m
    z = (o_ref[...] - m) / jnp.sqrt(v + 1e-5) * g_ref[...] + be_ref[...]
    z = jnp.where(z >= 0, z, 0.1 * z)
    h = jnp.dot(z, w_ref[...], preferred_element_type=jnp.float32)
    hl_ref[...] = h[:, :128]
    hr_ref[...] = h[:, 128:]
    ai_ref[...] = jnp.sum(h * atti_ref[...], axis=1, keepdims=True)
    aj_ref[...] = jnp.sum(h * attj_ref[...], axis=1, keepdims=True)


def _project(z, W, att, stats=None, g=None, be=None):
    n, d = z.shape
    emb = W.shape[0]
    attf = att.reshape(2 * emb)
    atti = attf[:emb].reshape(1, emb)
    attj = attf[emb:].reshape(1, emb)
    out_shape = [
        jax.ShapeDtypeStruct((n, 128), jnp.float32),
        jax.ShapeDtypeStruct((n, 128), jnp.float32),
        jax.ShapeDtypeStruct((n, 1), jnp.float32),
        jax.ShapeDtypeStruct((n, 1), jnp.float32),
    ]
    out_specs = [
        pl.BlockSpec((BLK, 128), lambda i: (i, 0)),
        pl.BlockSpec((BLK, 128), lambda i: (i, 0)),
        pl.BlockSpec((BLK, 1), lambda i: (i, 0)),
        pl.BlockSpec((BLK, 1), lambda i: (i, 0)),
    ]
    if stats is None:
        hl, hr, ai, aj = pl.pallas_call(
            _proj_kernel,
            grid=(n // BLK,),
            in_specs=[
                pl.BlockSpec((BLK, d), lambda i: (i, 0)),
                pl.BlockSpec((d, emb), lambda i: (0, 0)),
                pl.BlockSpec((1, emb), lambda i: (0, 0)),
                pl.BlockSpec((1, emb), lambda i: (0, 0)),
            ],
            out_specs=out_specs,
            out_shape=out_shape,
        )(z, W.T, atti, attj)
    else:
        hl, hr, ai, aj = pl.pallas_call(
            _proj_bn_kernel,
            grid=(n // BLK,),
            in_specs=[
                pl.BlockSpec((BLK, d), lambda i: (i, 0)),
                pl.BlockSpec((2, emb), lambda i: (0, 0)),
                pl.BlockSpec((1, emb), lambda i: (0, 0)),
                pl.BlockSpec((1, emb), lambda i: (0, 0)),
                pl.BlockSpec((d, emb), lambda i: (0, 0)),
                pl.BlockSpec((1, emb), lambda i: (0, 0)),
                pl.BlockSpec((1, emb), lambda i: (0, 0)),
            ],
            out_specs=out_specs,
            out_shape=out_shape,
        )(z, stats, g.reshape(1, emb), be.reshape(1, emb), W.T, atti, attj)
    return hl, hr, ai.reshape(n), aj.reshape(n)


def _stats_kernel(al_ref, ar_ref, b_ref, out_ref, st_ref, acc_ref):
    i = pl.program_id(0)
    o = jnp.concatenate([al_ref[...], ar_ref[...]], axis=1) + b_ref[...]
    o = jnp.maximum(o, 0.0)
    out_ref[...] = o
    rowid = lax.broadcasted_iota(jnp.int32, o.shape, 0) + i * BLK
    om = jnp.where(rowid < N, o, 0.0)
    part = jnp.stack([jnp.sum(om, axis=0), jnp.sum(om * om, axis=0)])

    @pl.when(i == 0)
    def _():
        acc_ref[...] = jnp.zeros_like(acc_ref)

    acc_ref[...] += part

    @pl.when(i == NBLK - 1)
    def _():
        st_ref[...] = acc_ref[...]


def _stats_relu(aggl, aggr, bias):
    emb = bias.shape[0]
    out, st = pl.pallas_call(
        _stats_kernel,
        grid=(NBLK,),
        in_specs=[
            pl.BlockSpec((BLK, 128), lambda i: (i, 0)),
            pl.BlockSpec((BLK, 128), lambda i: (i, 0)),
            pl.BlockSpec((1, emb), lambda i: (0, 0)),
        ],
        out_specs=[
            pl.BlockSpec((BLK, emb), lambda i: (i, 0)),
            pl.BlockSpec((2, emb), lambda i: (0, 0)),
        ],
        out_shape=[
            jax.ShapeDtypeStruct((NPAD, emb), jnp.float32),
            jax.ShapeDtypeStruct((2, emb), jnp.float32),
        ],
        scratch_shapes=[pltpu.VMEM((2, emb), jnp.float32)],
    )(aggl, aggr, bias.reshape(1, emb))
    return out, st


def _bn_apply_kernel(o_ref, st_ref, g_ref, be_ref, z_ref):
    m = st_ref[0, :] / N
    v = st_ref[1, :] / N - m * m
    z = (o_ref[...] - m) / jnp.sqrt(v + 1e-5) * g_ref[...] + be_ref[...]
    z_ref[...] = jnp.where(z >= 0, z, 0.1 * z)


def _bn_apply(out, stats, g, be):
    emb = g.shape[0]
    return pl.pallas_call(
        _bn_apply_kernel,
        grid=(NBLK,),
        in_specs=[
            pl.BlockSpec((BLK, emb), lambda i: (i, 0)),
            pl.BlockSpec((2, emb), lambda i: (0, 0)),
            pl.BlockSpec((1, emb), lambda i: (0, 0)),
            pl.BlockSpec((1, emb), lambda i: (0, 0)),
        ],
        out_specs=pl.BlockSpec((BLK, emb), lambda i: (i, 0)),
        out_shape=jax.ShapeDtypeStruct((NPAD, emb), jnp.float32),
    )(out, stats, g.reshape(1, emb), be.reshape(1, emb))


def _dec_kernel(a_ref, bb_ref, p1_ref, p2_ref, p1t_ref, y_ref):
    t = jnp.dot(a_ref[...], p1_ref[...], preferred_element_type=jnp.float32)
    t = jnp.dot(t, p2_ref[...], preferred_element_type=jnp.float32)
    mm = jnp.dot(t, p1t_ref[...], preferred_element_type=jnp.float32)
    y_ref[...] = jnp.sum(mm * bb_ref[...], axis=1, keepdims=True)


def _decode(a, bb, P1, P2):
    b = a.shape[0]
    return pl.pallas_call(
        _dec_kernel,
        out_shape=jax.ShapeDtypeStruct((b, 1), jnp.float32),
    )(a, bb, P1, P2, P1.T)


def _seg_combine(sv, val, ks_buf, vs_buf, is_max):
    ks, vs = plsc.sort_key_val(sv, val)
    ks_buf[...] = ks
    it = lax.iota(jnp.int32, 16)
    for sh in (1, 2, 4, 8):
        vs_buf[...] = vs
        idx = jnp.minimum(it + sh, 15)
        kg = plsc.load_gather(ks_buf, [idx])
        vg = plsc.load_gather(vs_buf, [idx])
        same = (kg == ks) & (it + sh <= 15)
        if is_max:
            vs = jnp.where(same, jnp.maximum(vs, vg), vs)
        else:
            vs = jnp.where(same, vs + vg, vs)
    prev = plsc.load_gather(ks_buf, [jnp.maximum(it - 1, 0)])
    first = (it == 0) | (ks != prev)
    return ks, vs, first


def _sc_compiler_params():
    cp = pltpu.CompilerParams()
    if "needs_layout_passes" in pltpu.CompilerParams.__dataclass_fields__:
        cp = dataclasses.replace(cp, needs_layout_passes=False)
    return cp


def _sc_weights(ai, aj, src16, dst16):
    mesh = plsc.VectorSubcoreMesh(core_axis_name="c", subcore_axis_name="s",
                                  num_cores=1, num_subcores=NTILE)

    @functools.partial(
        pl.kernel,
        out_type=jax.ShapeDtypeStruct((NTILE, T), jnp.float32),
        mesh=mesh,
        compiler_params=_sc_compiler_params(),
        scratch_types=[
            pltpu.VMEM((NPAD,), jnp.float32),
            pltpu.VMEM((NPAD,), jnp.float32),
            pltpu.VMEM((NPAD,), jnp.float32),
            pltpu.VMEM((NPAD,), jnp.float32),
            pltpu.VMEM((T,), jnp.int32),
            pltpu.VMEM((T,), jnp.int32),
            pltpu.VMEM((T,), jnp.float32),
            pltpu.VMEM((16,), jnp.int32),
            pltpu.VMEM((16,), jnp.float32),
            pltpu.VMEM((SLICE,), jnp.float32),
            pltpu.VMEM((SLICE,), jnp.float32),
            pltpu.VMEM_SHARED((NTILE, NPAD), jnp.float32),
            pltpu.VMEM_SHARED((NPAD,), jnp.float32),
        ],
    )
    def k(ai_hbm, aj_hbm, src_hbm, dst_hbm, w_hbm,
          ai_v, aj_v, m_v, s_v, src_v, dst_v, w_v,
          ks_buf, vs_buf, tmp, red, part, full):
        s = lax.axis_index("s")
        base = s * SLICE

        pltpu.sync_copy(ai_hbm, ai_v)
        pltpu.sync_copy(aj_hbm, aj_v)
        pltpu.sync_copy(src_hbm.at[s], src_v)
        pltpu.sync_copy(dst_hbm.at[s], dst_v)

        zero16 = jnp.zeros((16,), jnp.float32)
        neginf16 = jnp.full((16,), -jnp.inf, jnp.float32)

        @pl.loop(0, NPAD, step=16)
        def _(i):
            m_v[pl.ds(i, 16)] = neginf16
            s_v[pl.ds(i, 16)] = zero16

        def alpha_at(e0):
            sv = src_v[pl.ds(e0, 16)]
            dv = dst_v[pl.ds(e0, 16)]
            a1 = plsc.load_gather(ai_v, [dv])
            a2 = plsc.load_gather(aj_v, [sv])
            al = a1 + a2
            al = jnp.where(al >= 0.0, al, 0.2 * al)
            return sv, al

        @pl.loop(0, T, step=16)
        def _(e0):
            sv, al = alpha_at(e0)
            w_v[pl.ds(e0, 16)] = al
            ks, vs, first = _seg_combine(sv, al, ks_buf, vs_buf, is_max=True)
            cur = plsc.load_gather(m_v, [ks])
            plsc.store_scatter(m_v, [ks], jnp.maximum(cur, vs), mask=first)

        pltpu.sync_copy(m_v, part.at[s])
        plsc.subcore_barrier()
        pltpu.sync_copy(part.at[0, pl.ds(base, SLICE)], red)
        for t in range(1, NTILE):
            pltpu.sync_copy(part.at[t, pl.ds(base, SLICE)], tmp)

            @pl.loop(0, SLICE, step=16)
            def _(i):
                red[pl.ds(i, 16)] = jnp.maximum(red[pl.ds(i, 16)],
                                                tmp[pl.ds(i, 16)])

        pltpu.sync_copy(red, full.at[pl.ds(base, SLICE)])
        plsc.subcore_barrier()
        pltpu.sync_copy(full, m_v)

        @pl.loop(0, T, step=16)
        def _(e0):
            sv = src_v[pl.ds(e0, 16)]
            al = w_v[pl.ds(e0, 16)]
            am = plsc.load_gather(m_v, [sv])
            p = jnp.exp(al - am)
            ks, vs, first = _seg_combine(sv, p, ks_buf, vs_buf, is_max=False)
            cur = plsc.load_gather(s_v, [ks])
            plsc.store_scatter(s_v, [ks], cur + vs, mask=first)

        pltpu.sync_copy(s_v, part.at[s])
        plsc.subcore_barrier()
        pltpu.sync_copy(part.at[0, pl.ds(base, SLICE)], red)
        for t in range(1, NTILE):
            pltpu.sync_copy(part.at[t, pl.ds(base, SLICE)], tmp)

            @pl.loop(0, SLICE, step=16)
            def _(i):
                red[pl.ds(i, 16)] = red[pl.ds(i, 16)] + tmp[pl.ds(i, 16)]

        @pl.loop(0, SLICE, step=16)
        def _(i):
            red[pl.ds(i, 16)] = 1.0 / (red[pl.ds(i, 16)] + 1e-16)

        pltpu.sync_copy(red, full.at[pl.ds(base, SLICE)])
        plsc.subcore_barrier()
        pltpu.sync_copy(full, s_v)

        @pl.loop(0, T, step=16)
        def _(e0):
            sv = src_v[pl.ds(e0, 16)]
            al = w_v[pl.ds(e0, 16)]
            am = plsc.load_gather(m_v, [sv])
            rr = plsc.load_gather(s_v, [sv])
            w_v[pl.ds(e0, 16)] = jnp.exp(al - am) * rr

        pltpu.sync_copy(w_v, w_hbm.at[s])

    return k(ai, aj, src16, dst16)


def _sc_aggregate2(hstack, w16, src16, dst16):
    mesh = plsc.VectorSubcoreMesh(core_axis_name="c", subcore_axis_name="s",
                                  num_cores=2, num_subcores=NTILE)

    @functools.partial(
        pl.kernel,
        out_type=jax.ShapeDtypeStruct((2 * NPAD, 128), jnp.float32),
        mesh=mesh,
        compiler_params=_sc_compiler_params(),
        scratch_types=[
            pltpu.VMEM((CH3, 128), jnp.float32),
            pltpu.VMEM((CH3, 128), jnp.float32),
            pltpu.VMEM((SB,), jnp.float32),
            pltpu.VMEM((SB,), jnp.int32),
            pltpu.VMEM((SB,), jnp.int32),
            pltpu.SemaphoreType.DMA,
            pltpu.SemaphoreType.DMA,
            pltpu.VMEM_SHARED((NPAD, 128), jnp.float32),
        ],
    )
    def k(h_hbm, w_hbm, src_hbm, dst_hbm, out_hbm,
          rows_a, rows_b, w_t, dst_t, src_t, sem_a, sem_b, acc):
        c = lax.axis_index("c")
        s = lax.axis_index("s")
        base = s * SLICE
        hoff = c * NPAD

        zero16 = jnp.zeros((16,), jnp.float32)

        @pl.loop(0, ZB)
        def _(rr):
            for q in range(8):
                rows_a[rr, pl.ds(q * 16, 16)] = zero16

        for b in range(SLICE // ZB):
            pltpu.sync_copy(rows_a.at[pl.ds(0, ZB)],
                            acc.at[pl.ds(base + b * ZB, ZB)])

        plsc.subcore_barrier()

        off16 = jnp.full((16,), hoff, jnp.int32)

        def gather(e0, rows, sem):
            pltpu.make_async_copy(h_hbm.at[src_t.at[pl.ds(e0, CH3)]],
                                  rows, sem).start()

        def finish(e0, rows, sem):
            pltpu.make_async_copy(h_hbm.at[src_t.at[pl.ds(e0, CH3)]],
                                  rows, sem).wait()

            @pl.loop(0, CH3, step=16)
            def _(r0):
                wv = w_t[pl.ds(e0 + r0, 16)]
                for q1 in range(16):
                    wb = jnp.full((16,), wv[q1], jnp.float32)
                    for q2 in range(8):
                        sl = (r0 + q1, pl.ds(q2 * 16, 16))
                        rows[sl] = rows[sl] * wb

            pltpu.sync_copy(rows, acc.at[dst_t.at[pl.ds(e0, CH3)]], add=True)

        @pl.loop(0, NSB)
        def _(bi):
            b0 = bi * SB
            pltpu.sync_copy(w_hbm.at[s, pl.ds(b0, SB)], w_t)
            pltpu.sync_copy(dst_hbm.at[s, pl.ds(b0, SB)], dst_t)
            pltpu.sync_copy(src_hbm.at[s, pl.ds(b0, SB)], src_t)

            @pl.loop(0, SB, step=16)
            def _(i):
                src_t[pl.ds(i, 16)] = src_t[pl.ds(i, 16)] + off16

            gather(0, rows_a, sem_a)

            @pl.loop(0, NCB, step=2)
            def _(j):
                e0 = j * CH3
                gather(e0 + CH3, rows_b, sem_b)
                finish(e0, rows_a, sem_a)

                @pl.when(j + 2 < NCB)
                def _():
                    gather(e0 + 2 * CH3, rows_a, sem_a)

                finish(e0 + CH3, rows_b, sem_b)

        plsc.subcore_barrier()
        pltpu.sync_copy(acc.at[pl.ds(base, SLICE)],
                        out_hbm.at[pl.ds(hoff + base, SLICE)])

    return k(hstack, w16, src16, dst16)


def _gat_layer(z, W, att, bias, src16, dst16, stats=None, g=None, be=None):
    hl, hr, ai, aj = _project(z, W, att, stats, g, be)
    w16 = _sc_weights(ai, aj, src16, dst16)
    agg = _sc_aggregate2(jnp.concatenate([hl, hr], axis=0),
                         w16, src16, dst16)
    return agg[:NPAD], agg[NPAD:]


@jax.jit
def _forward_impl(x, edge_index, drug_index, W1, att1, b1, W2, att2, b2,
                  W3, att3, b3, g1, be1, g2, be2, g3, be3, P1, P2):
    loops = jnp.arange(N, dtype=edge_index.dtype)
    pad = jnp.full((EPAD - EHAT,), NPAD - 1, dtype=edge_index.dtype)
    src = jnp.concatenate([edge_index[0], loops, pad]).astype(jnp.int32)
    dst = jnp.concatenate([edge_index[1], loops, pad]).astype(jnp.int32)
    src16 = src.reshape(NTILE, T)
    dst16 = dst.reshape(NTILE, T)

    xp = jnp.pad(x, ((0, NPAD - N), (0, 0)))

    al, ar = _gat_layer(xp, W1, att1, b1, src16, dst16)
    out, st = _stats_relu(al, ar, b1)
    al, ar = _gat_layer(out, W2, att2, b2, src16, dst16, st, g1, be1)
    out, st = _stats_relu(al, ar, b2)
    al, ar = _gat_layer(out, W3, att3, b3, src16, dst16, st, g2, be2)
    out, st = _stats_relu(al, ar, b3)
    z3 = _bn_apply(out, st, g3, be3)

    di = drug_index.reshape(-1, 2)
    ia = (di[:, 0] - 1) % N
    ib = (di[:, 1] - 1) % N
    a = z3[ia]
    bb = z3[ib]
    return _decode(a, bb, P1, P2)


def kernel(x, edge_index, drug_index, W1, att1, b1, W2, att2, b2, W3, att3,
           b3, g1, be1, g2, be2, g3, be3, P1, P2):
    return _forward_impl(x, edge_index, drug_index, W1, att1, b1, W2, att2,
                         b2, W3, att3, b3, g1, be1, g2, be2, g3, be3, P1, P2)


# speedup vs baseline: 18.0967x; 1.0198x over previous
import dataclasses
import functools

import jax
import jax.numpy as jnp
from jax import lax
from jax.experimental import pallas as pl
from jax.experimental.pallas import tpu as pltpu
from jax.experimental.pallas import tpu_sc as plsc

N = 10000
NPAD = 10240
EHAT = 330000
NTILE = 16
CH3 = 128
ZB = 128
T = 20736
SB = 2304
NSB = T // SB
NCB = SB // CH3
EPAD = NTILE * T
NCHUNK3 = T // CH3
SLICE = NPAD // NTILE


BLK = 1024
NBLK = NPAD // BLK


def _proj_kernel(z_ref, w_ref, atti_ref, attj_ref,
                 hl_ref, hr_ref, ai_ref, aj_ref):
    h = jnp.dot(z_ref[...], w_ref[...], preferred_element_type=jnp.float32)
    hl_ref[...] = h[:, :128]
    hr_ref[...] = h[:, 128:]
    ai_ref[...] = jnp.sum(h * atti_ref[...], axis=1, keepdims=True)
    aj_ref[...] = jnp.sum(h * attj_ref[...], axis=1, keepdims=True)


def _proj_bn_kernel(o_ref, st_ref, g_ref, be_ref, w_ref, atti_ref, attj_ref,
                    hl_ref, hr_ref, ai_ref, aj_ref):
    m = st_ref[0, :] / N
    v = st_ref[1, :] / N - m * m
    z = (o_ref[...] - m) / jnp.sqrt(v + 1e-5) * g_ref[...] + be_ref[...]
    z = jnp.where(z >= 0, z, 0.1 * z)
    h = jnp.dot(z, w_ref[...], preferred_element_type=jnp.float32)
    hl_ref[...] = h[:, :128]
    hr_ref[...] = h[:, 128:]
    ai_ref[...] = jnp.sum(h * atti_ref[...], axis=1, keepdims=True)
    aj_ref[...] = jnp.sum(h * attj_ref[...], axis=1, keepdims=True)


def _project(z, W, att, stats=None, g=None, be=None):
    n, d = z.shape
    emb = W.shape[0]
    attf = att.reshape(2 * emb)
    atti = attf[:emb].reshape(1, emb)
    attj = attf[emb:].reshape(1, emb)
    out_shape = [
        jax.ShapeDtypeStruct((n, 128), jnp.float32),
        jax.ShapeDtypeStruct((n, 128), jnp.float32),
        jax.ShapeDtypeStruct((n, 1), jnp.float32),
        jax.ShapeDtypeStruct((n, 1), jnp.float32),
    ]
    out_specs = [
        pl.BlockSpec((BLK, 128), lambda i: (i, 0)),
        pl.BlockSpec((BLK, 128), lambda i: (i, 0)),
        pl.BlockSpec((BLK, 1), lambda i: (i, 0)),
        pl.BlockSpec((BLK, 1), lambda i: (i, 0)),
    ]
    if stats is None:
        hl, hr, ai, aj = pl.pallas_call(
            _proj_kernel,
            grid=(n // BLK,),
            in_specs=[
                pl.BlockSpec((BLK, d), lambda i: (i, 0)),
                pl.BlockSpec((d, emb), lambda i: (0, 0)),
                pl.BlockSpec((1, emb), lambda i: (0, 0)),
                pl.BlockSpec((1, emb), lambda i: (0, 0)),
            ],
            out_specs=out_specs,
            out_shape=out_shape,
        )(z, W.T, atti, attj)
    else:
        hl, hr, ai, aj = pl.pallas_call(
            _proj_bn_kernel,
            grid=(n // BLK,),
            in_specs=[
                pl.BlockSpec((BLK, d), lambda i: (i, 0)),
                pl.BlockSpec((2, emb), lambda i: (0, 0)),
                pl.BlockSpec((1, emb), lambda i: (0, 0)),
                pl.BlockSpec((1, emb), lambda i: (0, 0)),
                pl.BlockSpec((d, emb), lambda i: (0, 0)),
                pl.BlockSpec((1, emb), lambda i: (0, 0)),
                pl.BlockSpec((1, emb), lambda i: (0, 0)),
            ],
            out_specs=out_specs,
            out_shape=out_shape,
        )(z, stats, g.reshape(1, emb), be.reshape(1, emb), W.T, atti, attj)
    return hl, hr, ai.reshape(n), aj.reshape(n)


def _stats_kernel(al_ref, ar_ref, b_ref, out_ref, st_ref, acc_ref):
    i = pl.program_id(0)
    o = jnp.concatenate([al_ref[...], ar_ref[...]], axis=1) + b_ref[...]
    o = jnp.maximum(o, 0.0)
    out_ref[...] = o
    rowid = lax.broadcasted_iota(jnp.int32, o.shape, 0) + i * BLK
    om = jnp.where(rowid < N, o, 0.0)
    part = jnp.stack([jnp.sum(om, axis=0), jnp.sum(om * om, axis=0)])

    @pl.when(i == 0)
    def _():
        acc_ref[...] = jnp.zeros_like(acc_ref)

    acc_ref[...] += part

    @pl.when(i == NBLK - 1)
    def _():
        st_ref[...] = acc_ref[...]


def _stats_relu(aggl, aggr, bias):
    emb = bias.shape[0]
    out, st = pl.pallas_call(
        _stats_kernel,
        grid=(NBLK,),
        in_specs=[
            pl.BlockSpec((BLK, 128), lambda i: (i, 0)),
            pl.BlockSpec((BLK, 128), lambda i: (i, 0)),
            pl.BlockSpec((1, emb), lambda i: (0, 0)),
        ],
        out_specs=[
            pl.BlockSpec((BLK, emb), lambda i: (i, 0)),
            pl.BlockSpec((2, emb), lambda i: (0, 0)),
        ],
        out_shape=[
            jax.ShapeDtypeStruct((NPAD, emb), jnp.float32),
            jax.ShapeDtypeStruct((2, emb), jnp.float32),
        ],
        scratch_shapes=[pltpu.VMEM((2, emb), jnp.float32)],
    )(aggl, aggr, bias.reshape(1, emb))
    return out, st


def _bn_apply_kernel(o_ref, st_ref, g_ref, be_ref, z_ref):
    m = st_ref[0, :] / N
    v = st_ref[1, :] / N - m * m
    z = (o_ref[...] - m) / jnp.sqrt(v + 1e-5) * g_ref[...] + be_ref[...]
    z_ref[...] = jnp.where(z >= 0, z, 0.1 * z)


def _bn_apply(out, stats, g, be):
    emb = g.shape[0]
    return pl.pallas_call(
        _bn_apply_kernel,
        grid=(NBLK,),
        in_specs=[
            pl.BlockSpec((BLK, emb), lambda i: (i, 0)),
            pl.BlockSpec((2, emb), lambda i: (0, 0)),
            pl.BlockSpec((1, emb), lambda i: (0, 0)),
            pl.BlockSpec((1, emb), lambda i: (0, 0)),
        ],
        out_specs=pl.BlockSpec((BLK, emb), lambda i: (i, 0)),
        out_shape=jax.ShapeDtypeStruct((NPAD, emb), jnp.float32),
    )(out, stats, g.reshape(1, emb), be.reshape(1, emb))


def _dec_kernel(a_ref, bb_ref, p1_ref, p2_ref, p1t_ref, y_ref):
    t = jnp.dot(a_ref[...], p1_ref[...], preferred_element_type=jnp.float32)
    t = jnp.dot(t, p2_ref[...], preferred_element_type=jnp.float32)
    mm = jnp.dot(t, p1t_ref[...], preferred_element_type=jnp.float32)
    y_ref[...] = jnp.sum(mm * bb_ref[...], axis=1, keepdims=True)


def _decode(a, bb, P1, P2):
    b = a.shape[0]
    return pl.pallas_call(
        _dec_kernel,
        out_shape=jax.ShapeDtypeStruct((b, 1), jnp.float32),
    )(a, bb, P1, P2, P1.T)


def _seg_combine(sv, val, ks_buf, vs_buf, is_max):
    ks, vs = plsc.sort_key_val(sv, val)
    ks_buf[...] = ks
    it = lax.iota(jnp.int32, 16)
    for sh in (1, 2, 4, 8):
        vs_buf[...] = vs
        idx = jnp.minimum(it + sh, 15)
        kg = plsc.load_gather(ks_buf, [idx])
        vg = plsc.load_gather(vs_buf, [idx])
        same = (kg == ks) & (it + sh <= 15)
        if is_max:
            vs = jnp.where(same, jnp.maximum(vs, vg), vs)
        else:
            vs = jnp.where(same, vs + vg, vs)
    prev = plsc.load_gather(ks_buf, [jnp.maximum(it - 1, 0)])
    first = (it == 0) | (ks != prev)
    return ks, vs, first


def _sc_compiler_params():
    cp = pltpu.CompilerParams()
    if "needs_layout_passes" in pltpu.CompilerParams.__dataclass_fields__:
        cp = dataclasses.replace(cp, needs_layout_passes=False)
    return cp


def _sc_weights(ai, aj, src16, dst16):
    mesh = plsc.VectorSubcoreMesh(core_axis_name="c", subcore_axis_name="s",
                                  num_cores=1, num_subcores=NTILE)

    @functools.partial(
        pl.kernel,
        out_type=jax.ShapeDtypeStruct((NTILE, T), jnp.float32),
        mesh=mesh,
        compiler_params=_sc_compiler_params(),
        scratch_types=[
            pltpu.VMEM((NPAD,), jnp.float32),
            pltpu.VMEM((NPAD,), jnp.float32),
            pltpu.VMEM((NPAD,), jnp.float32),
            pltpu.VMEM((NPAD,), jnp.float32),
            pltpu.VMEM((T,), jnp.int32),
            pltpu.VMEM((T,), jnp.int32),
            pltpu.VMEM((T,), jnp.float32),
            pltpu.VMEM((16,), jnp.int32),
            pltpu.VMEM((16,), jnp.float32),
            pltpu.VMEM((SLICE,), jnp.float32),
            pltpu.VMEM((SLICE,), jnp.float32),
            pltpu.VMEM_SHARED((NTILE, NPAD), jnp.float32),
            pltpu.VMEM_SHARED((NPAD,), jnp.float32),
        ],
    )
    def k(ai_hbm, aj_hbm, src_hbm, dst_hbm, w_hbm,
          ai_v, aj_v, m_v, s_v, src_v, dst_v, w_v,
          ks_buf, vs_buf, tmp, red, part, full):
        s = lax.axis_index("s")
        base = s * SLICE

        pltpu.sync_copy(ai_hbm, ai_v)
        pltpu.sync_copy(aj_hbm, aj_v)
        pltpu.sync_copy(src_hbm.at[s], src_v)
        pltpu.sync_copy(dst_hbm.at[s], dst_v)

        zero16 = jnp.zeros((16,), jnp.float32)
        neginf16 = jnp.full((16,), -jnp.inf, jnp.float32)

        @pl.loop(0, NPAD, step=16)
        def _(i):
            m_v[pl.ds(i, 16)] = neginf16
            s_v[pl.ds(i, 16)] = zero16

        def alpha_at(e0):
            sv = src_v[pl.ds(e0, 16)]
            dv = dst_v[pl.ds(e0, 16)]
            a1 = plsc.load_gather(ai_v, [dv])
            a2 = plsc.load_gather(aj_v, [sv])
            al = a1 + a2
            al = jnp.where(al >= 0.0, al, 0.2 * al)
            return sv, al

        @pl.loop(0, T, step=16)
        def _(e0):
            sv, al = alpha_at(e0)
            w_v[pl.ds(e0, 16)] = al
            ks, vs, first = _seg_combine(sv, al, ks_buf, vs_buf, is_max=True)
            cur = plsc.load_gather(m_v, [ks])
            plsc.store_scatter(m_v, [ks], jnp.maximum(cur, vs), mask=first)

        pltpu.sync_copy(m_v, part.at[s])
        plsc.subcore_barrier()
        pltpu.sync_copy(part.at[0, pl.ds(base, SLICE)], red)
        for t in range(1, NTILE):
            pltpu.sync_copy(part.at[t, pl.ds(base, SLICE)], tmp)

            @pl.loop(0, SLICE, step=16)
            def _(i):
                red[pl.ds(i, 16)] = jnp.maximum(red[pl.ds(i, 16)],
                                                tmp[pl.ds(i, 16)])

        pltpu.sync_copy(red, full.at[pl.ds(base, SLICE)])
        plsc.subcore_barrier()
        pltpu.sync_copy(full, m_v)

        @pl.loop(0, T, step=16)
        def _(e0):
            sv = src_v[pl.ds(e0, 16)]
            al = w_v[pl.ds(e0, 16)]
            am = plsc.load_gather(m_v, [sv])
            p = jnp.exp(al - am)
            ks, vs, first = _seg_combine(sv, p, ks_buf, vs_buf, is_max=False)
            cur = plsc.load_gather(s_v, [ks])
            plsc.store_scatter(s_v, [ks], cur + vs, mask=first)

        pltpu.sync_copy(s_v, part.at[s])
        plsc.subcore_barrier()
        pltpu.sync_copy(part.at[0, pl.ds(base, SLICE)], red)
        for t in range(1, NTILE):
            pltpu.sync_copy(part.at[t, pl.ds(base, SLICE)], tmp)

            @pl.loop(0, SLICE, step=16)
            def _(i):
                red[pl.ds(i, 16)] = red[pl.ds(i, 16)] + tmp[pl.ds(i, 16)]

        @pl.loop(0, SLICE, step=16)
        def _(i):
            red[pl.ds(i, 16)] = 1.0 / (red[pl.ds(i, 16)] + 1e-16)

        pltpu.sync_copy(red, full.at[pl.ds(base, SLICE)])
        plsc.subcore_barrier()
        pltpu.sync_copy(full, s_v)

        @pl.loop(0, T, step=16)
        def _(e0):
            sv = src_v[pl.ds(e0, 16)]
            al = w_v[pl.ds(e0, 16)]
            am = plsc.load_gather(m_v, [sv])
            rr = plsc.load_gather(s_v, [sv])
            w_v[pl.ds(e0, 16)] = jnp.exp(al - am) * rr

        pltpu.sync_copy(w_v, w_hbm.at[s])

    return k(ai, aj, src16, dst16)


def _sc_aggregate2(hstack, w16, src16, dst16):
    mesh = plsc.VectorSubcoreMesh(core_axis_name="c", subcore_axis_name="s",
                                  num_cores=2, num_subcores=NTILE)

    @functools.partial(
        pl.kernel,
        out_type=jax.ShapeDtypeStruct((2 * NPAD, 128), jnp.float32),
        mesh=mesh,
        compiler_params=_sc_compiler_params(),
        scratch_types=[
            pltpu.VMEM((CH3, 128), jnp.float32),
            pltpu.VMEM((CH3, 128), jnp.float32),
            pltpu.VMEM((SB,), jnp.float32),
            pltpu.VMEM((SB,), jnp.int32),
            pltpu.VMEM((SB,), jnp.int32),
            pltpu.VMEM((SB,), jnp.float32),
            pltpu.VMEM((SB,), jnp.int32),
            pltpu.VMEM((SB,), jnp.int32),
            pltpu.SemaphoreType.DMA,
            pltpu.SemaphoreType.DMA,
            pltpu.SemaphoreType.DMA,
            pltpu.VMEM_SHARED((NPAD, 128), jnp.float32),
        ],
    )
    def k(h_hbm, w_hbm, src_hbm, dst_hbm, out_hbm,
          rows_a, rows_b, w_ta, dst_ta, src_ta, w_tb, dst_tb, src_tb,
          sem_a, sem_b, sem_s, acc):
        c = lax.axis_index("c")
        s = lax.axis_index("s")
        base = s * SLICE
        hoff = c * NPAD

        zero16 = jnp.zeros((16,), jnp.float32)

        @pl.loop(0, ZB)
        def _(rr):
            for q in range(8):
                rows_a[rr, pl.ds(q * 16, 16)] = zero16

        for b in range(SLICE // ZB):
            pltpu.sync_copy(rows_a.at[pl.ds(0, ZB)],
                            acc.at[pl.ds(base + b * ZB, ZB)])

        plsc.subcore_barrier()

        off16 = jnp.full((16,), hoff, jnp.int32)

        def stage_start(b0, w_t, dst_t, src_t):
            pltpu.make_async_copy(w_hbm.at[s, pl.ds(b0, SB)], w_t,
                                  sem_s).start()
            pltpu.make_async_copy(dst_hbm.at[s, pl.ds(b0, SB)], dst_t,
                                  sem_s).start()
            pltpu.make_async_copy(src_hbm.at[s, pl.ds(b0, SB)], src_t,
                                  sem_s).start()

        def stage_wait(b0, w_t, dst_t, src_t):
            pltpu.make_async_copy(w_hbm.at[s, pl.ds(b0, SB)], w_t,
                                  sem_s).wait()
            pltpu.make_async_copy(dst_hbm.at[s, pl.ds(b0, SB)], dst_t,
                                  sem_s).wait()
            pltpu.make_async_copy(src_hbm.at[s, pl.ds(b0, SB)], src_t,
                                  sem_s).wait()

            @pl.loop(0, SB, step=16)
            def _(i):
                src_t[pl.ds(i, 16)] = src_t[pl.ds(i, 16)] + off16

        def process(w_t, dst_t, src_t):
            def gather(e0, rows, sem):
                pltpu.make_async_copy(h_hbm.at[src_t.at[pl.ds(e0, CH3)]],
                                      rows, sem).start()

            def finish(e0, rows, sem):
                pltpu.make_async_copy(h_hbm.at[src_t.at[pl.ds(e0, CH3)]],
                                      rows, sem).wait()

                @pl.loop(0, CH3, step=16)
                def _(r0):
                    wv = w_t[pl.ds(e0 + r0, 16)]
                    for q1 in range(16):
                        wb = jnp.full((16,), wv[q1], jnp.float32)
                        for q2 in range(8):
                            sl = (r0 + q1, pl.ds(q2 * 16, 16))
                            rows[sl] = rows[sl] * wb

                pltpu.sync_copy(rows, acc.at[dst_t.at[pl.ds(e0, CH3)]],
                                add=True)

            gather(0, rows_a, sem_a)

            @pl.loop(0, NCB, step=2)
            def _(j):
                e0 = j * CH3
                gather(e0 + CH3, rows_b, sem_b)
                finish(e0, rows_a, sem_a)

                @pl.when(j + 2 < NCB)
                def _():
                    gather(e0 + 2 * CH3, rows_a, sem_a)

                finish(e0 + CH3, rows_b, sem_b)

        stage_start(0, w_ta, dst_ta, src_ta)
        stage_wait(0, w_ta, dst_ta, src_ta)

        @pl.loop(1, NSB, step=2)
        def _(bi):
            b0 = bi * SB
            stage_start(b0, w_tb, dst_tb, src_tb)
            process(w_ta, dst_ta, src_ta)
            stage_wait(b0, w_tb, dst_tb, src_tb)

            @pl.when(bi + 1 < NSB)
            def _():
                stage_start(b0 + SB, w_ta, dst_ta, src_ta)

            process(w_tb, dst_tb, src_tb)

            @pl.when(bi + 1 < NSB)
            def _():
                stage_wait(b0 + SB, w_ta, dst_ta, src_ta)

        process(w_ta, dst_ta, src_ta)

        plsc.subcore_barrier()
        pltpu.sync_copy(acc.at[pl.ds(base, SLICE)],
                        out_hbm.at[pl.ds(hoff + base, SLICE)])

    return k(hstack, w16, src16, dst16)


def _gat_layer(z, W, att, bias, src16, dst16, stats=None, g=None, be=None):
    hl, hr, ai, aj = _project(z, W, att, stats, g, be)
    w16 = _sc_weights(ai, aj, src16, dst16)
    agg = _sc_aggregate2(jnp.concatenate([hl, hr], axis=0),
                         w16, src16, dst16)
    return agg[:NPAD], agg[NPAD:]


@jax.jit
def _forward_impl(x, edge_index, drug_index, W1, att1, b1, W2, att2, b2,
                  W3, att3, b3, g1, be1, g2, be2, g3, be3, P1, P2):
    loops = jnp.arange(N, dtype=edge_index.dtype)
    pad = jnp.full((EPAD - EHAT,), NPAD - 1, dtype=edge_index.dtype)
    src = jnp.concatenate([edge_index[0], loops, pad]).astype(jnp.int32)
    dst = jnp.concatenate([edge_index[1], loops, pad]).astype(jnp.int32)
    src16 = src.reshape(NTILE, T)
    dst16 = dst.reshape(NTILE, T)

    xp = jnp.pad(x, ((0, NPAD - N), (0, 0)))

    al, ar = _gat_layer(xp, W1, att1, b1, src16, dst16)
    out, st = _stats_relu(al, ar, b1)
    al, ar = _gat_layer(out, W2, att2, b2, src16, dst16, st, g1, be1)
    out, st = _stats_relu(al, ar, b2)
    al, ar = _gat_layer(out, W3, att3, b3, src16, dst16, st, g2, be2)
    out, st = _stats_relu(al, ar, b3)
    z3 = _bn_apply(out, st, g3, be3)

    di = drug_index.reshape(-1, 2)
    ia = (di[:, 0] - 1) % N
    ib = (di[:, 1] - 1) % N
    a = z3[ia]
    bb = z3[ib]
    return _decode(a, bb, P1, P2)


def kernel(x, edge_index, drug_index, W1, att1, b1, W2, att2, b2, W3, att3,
           b3, g1, be1, g2, be2, g3, be3, P1, P2):
    return _forward_impl(x, edge_index, drug_index, W1, att1, b1, W2, att2,
                         b2, W3, att3, b3, g1, be1, g2, be2, g3, be3, P1, P2)
